# Initial kernel scaffold; baseline (speedup 1.0000x reference)
import jax
import jax.numpy as jnp
from jax.experimental import pallas as pl


def kernel(gene_feat, mech_feat, drug_feat, params, gm_src, gm_dst, dd_edge_index, gene_idx, drug_idx):
    raise NotImplementedError("write your pallas kernel here")


# speedup vs baseline: 11.1640x; 11.1640x over previous
import functools

import jax
import jax.numpy as jnp
from jax import lax
from jax.experimental import pallas as pl
from jax.experimental.pallas import tpu as pltpu
from jax.experimental.pallas import tpu_sc as plsc

NC = 2
NS = 16
NW = NC * NS
LEAK = 0.2


def _cdiv(a, b):
    return (a + b - 1) // b


def _ln(x, g, b):
    m = x.mean(-1, keepdims=True)
    v = ((x - m) ** 2).mean(-1, keepdims=True)
    return (x - m) / jnp.sqrt(v + 1e-5) * g + b


def _leaky(x):
    return jnp.where(x >= 0, x, LEAK * x)


def _eye(n):
    return (lax.broadcasted_iota(jnp.int32, (n, n), 0)
            == lax.broadcasted_iota(jnp.int32, (n, n), 1)).astype(jnp.float32)


def _mm(a, b):
    return jnp.dot(a, b, preferred_element_type=jnp.float32)


def _sc_build(nm, gm_src3, gm_dst3, dd_src3, dd_dst3, zc, za,
              c_out, a_out,
              src_v, dst_v, dsrc_v, ddst_v, acc_c, acc_a):
    kgm = src_v.shape[0]
    kdd = dsrc_v.shape[0]
    csl = acc_c.shape[0]
    arows = acc_a.shape[0]
    cid = lax.axis_index("c")
    sid = lax.axis_index("s")
    wid = sid * NC + cid
    ones16 = jnp.full((16,), 1.0, jnp.float32)

    lo_c = wid * csl
    pltpu.sync_copy(zc.at[pl.ds(pl.multiple_of(lo_c, 8), csl)], acc_c)

    def gm_blk(b, carry):
        pltpu.sync_copy(gm_src3.at[b], src_v)
        pltpu.sync_copy(gm_dst3.at[b], dst_v)

        def row(r, c2):
            for j in range(128 // 16):
                s = src_v[r, pl.ds(j * 16, 16)]
                d = dst_v[r, pl.ds(j * 16, 16)]
                loc = s * nm + d - lo_c
                msk = (loc >= 0) & (loc < csl)
                locc = jnp.clip(loc, 0, csl - 1)
                plsc.addupdate_scatter(acc_c, [locc], ones16, mask=msk)
            return c2

        return lax.fori_loop(0, kgm, row, carry)

    lax.fori_loop(0, NW, gm_blk, 0)
    pltpu.sync_copy(acc_c, c_out.at[pl.ds(pl.multiple_of(lo_c, 8), csl)])

    for p in range(2):
        lo_r = wid * (2 * arows) + p * arows
        pltpu.sync_copy(za, acc_a)

        def dd_blk(b, carry):
            pltpu.sync_copy(dd_src3.at[b], dsrc_v)
            pltpu.sync_copy(dd_dst3.at[b], ddst_v)

            def row(r, c2):
                for j in range(128 // 16):
                    s = dsrc_v[r, pl.ds(j * 16, 16)]
                    d = ddst_v[r, pl.ds(j * 16, 16)]
                    rr = d - lo_r
                    msk = (rr >= 0) & (rr < arows)
                    rrc = jnp.clip(rr, 0, arows - 1)
                    plsc.addupdate_scatter(acc_a, [rrc, s], ones16, mask=msk)
                return c2

            return lax.fori_loop(0, kdd, row, carry)

        lax.fori_loop(0, NW, dd_blk, 0)
        pltpu.sync_copy(acc_a, a_out.at[pl.ds(pl.multiple_of(lo_r, 8), arows)])


def _sc_gather(hgc, demb, gidx3, didx3,
               geneh_out, de_out,
               gidx_v, didx_v, rows_h, rows_d, sem):
    kb = gidx_v.shape[0]
    cid = lax.axis_index("c")
    sid = lax.axis_index("s")
    wid = sid * NC + cid
    pltpu.sync_copy(gidx3.at[wid], gidx_v)
    pltpu.sync_copy(didx3.at[wid], didx_v)
    for j in range(kb):
        base = pl.multiple_of(wid * (kb * 128) + j * 128, 8)
        pltpu.async_copy(hgc.at[gidx_v.at[j]], rows_h, sem).wait()
        pltpu.sync_copy(rows_h, geneh_out.at[pl.ds(base, 128)])
        pltpu.async_copy(demb.at[didx_v.at[j]], rows_d, sem).wait()
        pltpu.sync_copy(rows_d, de_out.at[pl.ds(base, 128)])


def _tc_gene_body(H, C, gf, cm, mech, gW, gb, glng, glnb, mW, mb, mlng,
                  mlnb, Wg2m, asg2m, adg2m, Wm2g, asm2g, adm2g, bm2g, lng0g,
                  lng0b, hgc_o, mnew_o, den_o):
    i = pl.program_id(0)
    NM = mech.shape[0]
    HID = gW.shape[1]
    R = gf.shape[0]
    x = gf[...]
    hg0 = _ln(_mm(x, gW[...]) + gb[...], glng[...], glnb[...])
    hm0 = _ln(_mm(mech[...], mW[...]) + mb[...], mlng[...], mlnb[...])
    cb = cm[...]
    eye = _eye(H)
    hs_g = _mm(hg0, Wg2m[...])
    es_g = (hs_g.reshape(R, H, C) * asg2m[...][None]).sum(-1)
    ed_m = ((_mm(hm0, Wg2m[...])).reshape(NM, H, C) * adg2m[...][None]).sum(-1)
    z = _leaky(es_g[:, None, :] + ed_m[None, :, :])
    wgt = cb[:, :, None] * jnp.exp(z)
    res = lax.dot_general(wgt.reshape(R, NM * H), hs_g,
                          (((0,), (0,)), ((), ())),
                          preferred_element_type=jnp.float32)
    mnew_add = (res.reshape(NM, H, H, C) * eye[None, :, :, None]).sum(2)
    den_add = wgt.sum(0)
    den_pad = jnp.concatenate(
        [den_add, jnp.zeros((NM, 128 - H), jnp.float32)], axis=1)

    @pl.when(i == 0)
    def _():
        mnew_o[...] = jnp.zeros((NM, HID), jnp.float32)
        den_o[...] = jnp.zeros((NM, 128), jnp.float32)

    mnew_o[...] += mnew_add.reshape(NM, HID)
    den_o[...] += den_pad
    hs_m = _mm(hm0, Wm2g[...])
    es_m = (hs_m.reshape(NM, H, C) * asm2g[...][None]).sum(-1)
    wd = (Wm2g[...].reshape(HID, H, C) * adm2g[...][None]).sum(-1)
    ed_g = _mm(hg0, wd)
    z2 = _leaky(es_m[None, :, :] + ed_g[:, None, :])
    w2 = cb[:, :, None] * jnp.exp(z2)
    al = w2 / (w2.sum(1)[:, None, :] + 1e-16)
    hs2 = (hs_m.reshape(NM, 1, H, C) * eye[None, :, :, None]).reshape(NM * H, HID)
    gnew = _mm(al.reshape(R, NM * H), hs2) + bm2g[...]
    hg1 = jax.nn.gelu(_ln(gnew + hg0, lng0g[...], lng0b[...]))
    hgc_o[...] = jnp.concatenate(
        [hg1, cb, jnp.zeros((R, 128 - NM), jnp.float32)], axis=1)


def _tc_drugmlp_body(df, dW1, db1, l1g, l1b, dW2, db2, l2g, l2b, h2_o):
    h1 = jax.nn.gelu(_ln(_mm(df[...], dW1[...]) + db1[...], l1g[...], l1b[...]))
    h2_o[...] = jax.nn.gelu(_ln(_mm(h1, dW2[...]) + db2[...], l2g[...], l2b[...]))


def _tc_drugout_body(ablk, h2f, h2b, sWl, sbl, sWr, slng, slnb,
                     doW, dob, demb_o):
    A = ablk[...]
    deg = A.sum(-1, keepdims=True)
    msg = _mm(A, h2f[...]) / jnp.maximum(deg, 1.0)
    h = h2b[...]
    hn = _mm(msg, sWl[...]) + sbl[...] + _mm(h, sWr[...])
    hd = jax.nn.gelu(_ln(h + hn, slng[...], slnb[...]))
    demb_o[...] = _mm(hd, doW[...]) + dob[...]


def _tc_decode_body(H, C, geneh, de, mnew_un, den, mech, mW, mb, mlng,
                    mlnb, g2m0b, lnm0g, lnm0b, Wm2g1, asm2g1, adm2g1, bm2g1,
                    lng1g, lng1b, goW, gob, protos, gmW1, gmb1, gmW2v, gmb2v,
                    Wbil, sc_o):
    NM = mech.shape[0]
    HID = mW.shape[1]
    OUT = goW.shape[1]
    P = geneh.shape[0]
    eye = _eye(H)
    hm0 = _ln(_mm(mech[...], mW[...]) + mb[...], mlng[...], mlnb[...])
    mnew0 = (mnew_un[...].reshape(NM, H, C)
             / (den[...][:, 0:H].reshape(NM, H, 1) + 1e-16)).reshape(NM, HID)
    hm1 = jax.nn.gelu(_ln(mnew0 + g2m0b[...] + hm0, lnm0g[...], lnm0b[...]))
    hs_m = _mm(hm1, Wm2g1[...])
    es_m = (hs_m.reshape(NM, H, C) * asm2g1[...][None]).sum(-1)
    wd = (Wm2g1[...].reshape(HID, H, C) * adm2g1[...][None]).sum(-1)
    geh = geneh[:, 0:HID]
    cg = geneh[:, HID:HID + NM]
    ed_g = _mm(geh, wd)
    z = _leaky(es_m[None, :, :] + ed_g[:, None, :])
    w3 = cg[:, :, None] * jnp.exp(z)
    al = w3 / (w3.sum(1)[:, None, :] + 1e-16)
    hs2 = (hs_m.reshape(NM, 1, H, C) * eye[None, :, :, None]).reshape(NM * H, HID)
    gnew = _mm(al.reshape(P, NM * H), hs2) + bm2g1[...]
    hg2 = jax.nn.gelu(_ln(gnew + geh, lng1g[...], lng1b[...]))
    ge = _mm(hg2, goW[...]) + gob[...]
    u = _mm(ge, gmW1[0:OUT, :])
    v = _mm(protos[...], gmW1[OUT:2 * OUT, :])
    gact = jax.nn.gelu(u[:, None, :] + v[None, :, :] + gmb1[...][None])
    gates = (gact * gmW2v[...][None]).sum(-1) + gmb2v[...]
    mx = gates.max(-1, keepdims=True)
    ex = jnp.exp(gates - mx)
    w = ex / ex.sum(-1, keepdims=True)
    gfin = ge + _mm(w, protos[...])
    sc_o[...] = (_mm(gfin, Wbil[...]) * de[...]).sum(-1)


def kernel(gene_feat, mech_feat, drug_feat, params, gm_src, gm_dst,
           dd_edge_index, gene_idx, drug_idx):
    p = params
    NG, GFD = gene_feat.shape
    NM, MFD = mech_feat.shape
    ND, DFD = drug_feat.shape
    EGM = gm_src.shape[0]
    EDD = dd_edge_index.shape[1]
    B = gene_idx.shape[0]
    HID = p['gW'].shape[1]
    OUT = p['goW'].shape[1]
    H, C = p['g2m0_as'].shape
    f32 = jnp.float32

    R = 256
    NGP = _cdiv(NG, R) * R
    NDP = _cdiv(ND, 128) * 128
    CPAD = _cdiv(NG * NM + 1, NW * 8) * NW * 8
    CSL = CPAD // NW
    AROWS = NDP // (2 * NW)

    v2 = lambda a: a.reshape(1, -1)
    gf_p = jnp.pad(gene_feat, ((0, NGP - NG), (0, 0)))
    df_p = jnp.pad(drug_feat, ((0, NDP - ND), (0, 0)))

    kgm = _cdiv(EGM, NW * 128)
    egm_p = NW * kgm * 128
    gm_src_p = jnp.pad(gm_src, (0, egm_p - EGM), constant_values=NG)
    gm_dst_p = jnp.pad(gm_dst, (0, egm_p - EGM))
    gm_src3 = gm_src_p.reshape(NW, kgm, 128).astype(jnp.int32)
    gm_dst3 = gm_dst_p.reshape(NW, kgm, 128).astype(jnp.int32)

    kdd = _cdiv(EDD, NW * 128)
    edd_p = NW * kdd * 128
    dd_src_p = jnp.pad(dd_edge_index[0], (0, edd_p - EDD))
    dd_dst_p = jnp.pad(dd_edge_index[1], (0, edd_p - EDD), constant_values=NDP - 1)
    dd_src3 = dd_src_p.reshape(NW, kdd, 128).astype(jnp.int32)
    dd_dst3 = dd_dst_p.reshape(NW, kdd, 128).astype(jnp.int32)

    kb = B // (NW * 128)
    gidx3 = gene_idx.reshape(NW, kb, 128).astype(jnp.int32)
    didx3 = drug_idx.reshape(NW, kb, 128).astype(jnp.int32)

    zc = jnp.zeros((CPAD,), f32)
    za = jnp.zeros((AROWS, NDP), f32)

    mesh = plsc.VectorSubcoreMesh(core_axis_name="c", subcore_axis_name="s",
                                  num_cores=NC, num_subcores=NS)

    sc_build = functools.partial(
        pl.kernel, mesh=mesh,
        compiler_params=pltpu.CompilerParams(needs_layout_passes=False),
        out_type=[jax.ShapeDtypeStruct((CPAD,), f32),
                  jax.ShapeDtypeStruct((NDP, NDP), f32)],
        scratch_types=[pltpu.VMEM((kgm, 128), jnp.int32),
                       pltpu.VMEM((kgm, 128), jnp.int32),
                       pltpu.VMEM((kdd, 128), jnp.int32),
                       pltpu.VMEM((kdd, 128), jnp.int32),
                       pltpu.VMEM((CSL,), f32),
                       pltpu.VMEM((AROWS, NDP), f32)],
    )(functools.partial(_sc_build, NM))
    c_flat, amat = sc_build(gm_src3, gm_dst3, dd_src3, dd_dst3, zc, za)

    cmat = jnp.pad(c_flat[:NG * NM].reshape(NG, NM), ((0, NGP - NG), (0, 0)))

    full = lambda shape: pl.BlockSpec(shape, lambda i: tuple(0 for _ in shape))
    h2 = pl.pallas_call(
        _tc_drugmlp_body,
        grid=(NDP // R,),
        in_specs=[pl.BlockSpec((R, DFD), lambda i: (i, 0)),
                  full((DFD, HID)), full((1, HID)), full((1, HID)), full((1, HID)),
                  full((HID, HID)), full((1, HID)), full((1, HID)), full((1, HID))],
        out_specs=pl.BlockSpec((R, HID), lambda i: (i, 0)),
        out_shape=jax.ShapeDtypeStruct((NDP, HID), f32),
    )(df_p, p['dW1'], v2(p['db1']), v2(p['dln1_g']), v2(p['dln1_b']),
      p['dW2'], v2(p['db2']), v2(p['dln2_g']), v2(p['dln2_b']))

    demb = pl.pallas_call(
        _tc_drugout_body,
        grid=(NDP // R,),
        in_specs=[pl.BlockSpec((R, NDP), lambda i: (i, 0)),
                  full((NDP, HID)),
                  pl.BlockSpec((R, HID), lambda i: (i, 0)),
                  full((HID, HID)), full((1, HID)), full((HID, HID)),
                  full((1, HID)), full((1, HID)), full((HID, OUT)), full((1, OUT))],
        out_specs=pl.BlockSpec((R, OUT), lambda i: (i, 0)),
        out_shape=jax.ShapeDtypeStruct((NDP, OUT), f32),
    )(amat, h2, h2,
      p['sWl'], v2(p['sbl']), p['sWr'], v2(p['sln_g']), v2(p['sln_b']),
      p['doW'], v2(p['dob']))

    hgc, mnew_un, den = pl.pallas_call(
        functools.partial(_tc_gene_body, H, C),
        grid=(NGP // R,),
        in_specs=[pl.BlockSpec((R, GFD), lambda i: (i, 0)),
                  pl.BlockSpec((R, NM), lambda i: (i, 0)),
                  full((NM, MFD)),
                  full((GFD, HID)), full((1, HID)), full((1, HID)), full((1, HID)),
                  full((MFD, HID)), full((1, HID)), full((1, HID)), full((1, HID)),
                  full((HID, HID)), full((H, C)), full((H, C)),
                  full((HID, HID)), full((H, C)), full((H, C)), full((1, HID)),
                  full((1, HID)), full((1, HID))],
        out_specs=[pl.BlockSpec((R, HID + 128), lambda i: (i, 0)),
                   full((NM, HID)), full((NM, 128))],
        out_shape=[jax.ShapeDtypeStruct((NGP, HID + 128), f32),
                   jax.ShapeDtypeStruct((NM, HID), f32),
                   jax.ShapeDtypeStruct((NM, 128), f32)],
    )(gf_p, cmat, mech_feat,
      p['gW'], v2(p['gb']), v2(p['g_ln_g']), v2(p['g_ln_b']),
      p['mW'], v2(p['mb']), v2(p['m_ln_g']), v2(p['m_ln_b']),
      p['g2m0_W'], p['g2m0_as'], p['g2m0_ad'],
      p['m2g0_W'], p['m2g0_as'], p['m2g0_ad'], v2(p['m2g0_b']),
      v2(p['lng0_g']), v2(p['lng0_b']))

    sc_gather = functools.partial(
        pl.kernel, mesh=mesh,
        compiler_params=pltpu.CompilerParams(needs_layout_passes=False),
        out_type=[jax.ShapeDtypeStruct((B, HID + 128), f32),
                  jax.ShapeDtypeStruct((B, OUT), f32)],
        scratch_types=[pltpu.VMEM((kb, 128), jnp.int32),
                       pltpu.VMEM((kb, 128), jnp.int32),
                       pltpu.VMEM((128, HID + 128), f32),
                       pltpu.VMEM((128, OUT), f32),
                       pltpu.SemaphoreType.DMA],
    )(_sc_gather)
    geneh, de = sc_gather(hgc, demb, gidx3, didx3)

    P = 256
    scores = pl.pallas_call(
        functools.partial(_tc_decode_body, H, C),
        grid=(B // P,),
        in_specs=[pl.BlockSpec((P, HID + 128), lambda i: (i, 0)),
                  pl.BlockSpec((P, OUT), lambda i: (i, 0)),
                  full((NM, HID)), full((NM, 128)), full((NM, MFD)),
                  full((MFD, HID)), full((1, HID)), full((1, HID)), full((1, HID)),
                  full((1, HID)), full((1, HID)), full((1, HID)),
                  full((HID, HID)), full((H, C)), full((H, C)), full((1, HID)),
                  full((1, HID)), full((1, HID)),
                  full((HID, OUT)), full((1, OUT)), full((NM, OUT)),
                  full((2 * OUT, OUT)), full((1, OUT)), full((1, OUT)),
                  full((1, NM)), full((OUT, OUT))],
        out_specs=pl.BlockSpec((P,), lambda i: (i,)),
        out_shape=jax.ShapeDtypeStruct((B,), f32),
    )(geneh, de, mnew_un, den, mech_feat,
      p['mW'], v2(p['mb']), v2(p['m_ln_g']), v2(p['m_ln_b']),
      v2(p['g2m0_b']), v2(p['lnm0_g']), v2(p['lnm0_b']),
      p['m2g1_W'], p['m2g1_as'], p['m2g1_ad'], v2(p['m2g1_b']),
      v2(p['lng1_g']), v2(p['lng1_b']),
      p['goW'], v2(p['gob']), p['protos'],
      p['gmW1'], v2(p['gmb1']), p['gmW2'].reshape(1, OUT),
      jnp.broadcast_to(p['gmb2'].reshape(1, 1), (1, NM)), p['Wbil'])

    return scores


# speedup vs baseline: 30.5527x; 2.7367x over previous
import functools

import jax
import jax.numpy as jnp
from jax import lax
from jax.experimental import pallas as pl
from jax.experimental.pallas import tpu as pltpu
from jax.experimental.pallas import tpu_sc as plsc

NC = 2
NS = 16
NW = NC * NS
LEAK = 0.2


def _cdiv(a, b):
    return (a + b - 1) // b


def _ln(x, g, b):
    m = x.mean(-1, keepdims=True)
    v = ((x - m) ** 2).mean(-1, keepdims=True)
    return (x - m) / jnp.sqrt(v + 1e-5) * g + b


def _leaky(x):
    return jnp.where(x >= 0, x, LEAK * x)


def _mm(a, b):
    return jnp.dot(a, b, preferred_element_type=jnp.float32)


def _iota2(shape, d):
    return lax.broadcasted_iota(jnp.int32, shape, d)


def _sel(shape, fn):
    return fn(_iota2(shape, 0), _iota2(shape, 1)).astype(jnp.float32)


def _flat_mh(a, T4, R8):
    return (R8 * jnp.dot(a, T4, preferred_element_type=jnp.float32)
            ).sum(0, keepdims=True)


def _sc_build(nm, gm_src2, gm_dst2, dd_src2, dd_dst2, zc, za,
              c_out, a_out,
              src_v, dst_v, dsrc_v, ddst_v, acc_c, acc_a):
    rg = gm_src2.shape[0]
    chg = src_v.shape[0]
    rd = dd_src2.shape[0]
    chd = dsrc_v.shape[0]
    csl = acc_c.shape[0]
    arows = acc_a.shape[0]
    cid = lax.axis_index("c")
    sid = lax.axis_index("s")
    wid = sid * NC + cid
    ones16 = jnp.full((16,), 1.0, jnp.float32)

    lo_c = wid * csl
    pltpu.sync_copy(zc.at[pl.ds(pl.multiple_of(lo_c, 8), csl)], acc_c)
    for t in range(rg // chg):
        pltpu.sync_copy(gm_src2.at[pl.ds(t * chg, chg)], src_v)
        pltpu.sync_copy(gm_dst2.at[pl.ds(t * chg, chg)], dst_v)

        def row(r, c2):
            for j in range(128 // 16):
                s = src_v[r, pl.ds(j * 16, 16)]
                d = dst_v[r, pl.ds(j * 16, 16)]
                loc = s * nm + d - lo_c
                msk = (loc >= 0) & (loc < csl)
                locc = jnp.clip(loc, 0, csl - 1)
                plsc.addupdate_scatter(acc_c, [locc], ones16, mask=msk)
            return c2

        lax.fori_loop(0, chg, row, 0)
    pltpu.sync_copy(acc_c, c_out.at[pl.ds(pl.multiple_of(lo_c, 8), csl)])

    for p in range(2):
        lo_r = wid * (2 * arows) + p * arows
        pltpu.sync_copy(za, acc_a)
        for t in range(rd // chd):
            pltpu.sync_copy(dd_src2.at[pl.ds(t * chd, chd)], dsrc_v)
            pltpu.sync_copy(dd_dst2.at[pl.ds(t * chd, chd)], ddst_v)

            def row2(r, c2):
                for j in range(128 // 16):
                    s = dsrc_v[r, pl.ds(j * 16, 16)]
                    d = ddst_v[r, pl.ds(j * 16, 16)]
                    rr = d - lo_r
                    msk = (rr >= 0) & (rr < arows)
                    rrc = jnp.clip(rr, 0, arows - 1)
                    plsc.addupdate_scatter(acc_a, [rrc, s], ones16, mask=msk)
                return c2

            lax.fori_loop(0, chd, row2, 0)
        pltpu.sync_copy(acc_a, a_out.at[pl.ds(pl.multiple_of(lo_r, 8), arows)])


def _sc_gather(hgc, demb, gidx3, didx3,
               geneh_out, de_out,
               gidx_v, didx_v, rows_h, rows_d, sem):
    kb = gidx_v.shape[0]
    cid = lax.axis_index("c")
    sid = lax.axis_index("s")
    wid = sid * NC + cid
    pltpu.sync_copy(gidx3.at[wid], gidx_v)
    pltpu.sync_copy(didx3.at[wid], didx_v)
    for j in range(kb):
        base = pl.multiple_of(wid * (kb * 128) + j * 128, 8)
        pltpu.async_copy(hgc.at[gidx_v.at[j]], rows_h, sem).wait()
        pltpu.sync_copy(rows_h, geneh_out.at[pl.ds(base, 128)])
        pltpu.async_copy(demb.at[didx_v.at[j]], rows_d, sem).wait()
        pltpu.sync_copy(rows_d, de_out.at[pl.ds(base, 128)])


def _tc_gene_body(H, C, gf, cm, mech, gW, gb, glng, glnb, mW, mb, mlng,
                  mlnb, Wg2m, asg2m, adg2m, Wm2g, asm2g, adm2g, bm2g, lng0g,
                  lng0b, hgc_o, mnew_o, den_o):
    i = pl.program_id(0)
    NM = mech.shape[0]
    HID = gW.shape[1]
    R = gf.shape[0]
    MH = NM * H
    bf16 = jnp.bfloat16
    f32 = jnp.float32
    x = gf[...]
    hg0 = _ln(_mm(x, gW[...]) + gb[...], glng[...], glnb[...])
    hm0 = _ln(_mm(mech[...], mW[...]) + mb[...], mlng[...], mlnb[...])
    cb = cm[...]
    G64 = _sel((HID, H), lambda r, c: (r // C) == c)
    T4 = _sel((H, MH), lambda r, c: (c % H) == r)
    T4T = _sel((MH, H), lambda r, c: (r % H) == c)
    R8 = _sel((NM, MH), lambda r, c: (c // H) == r)
    G4T = _sel((MH, NM), lambda r, c: (r // H) == c)
    M32 = _sel((MH, HID), lambda r, c: (r % H) == (c // C))
    hg0b = hg0.astype(bf16)
    hs_g = _mm(hg0b, Wg2m[...])
    es_g = _mm(hs_g * asg2m[...], G64)
    ed_m = _mm(_mm(hm0.astype(bf16), Wg2m[...]) * adg2m[...], G64)
    z32 = _leaky(_mm(es_g, T4) + _flat_mh(ed_m, T4, R8))
    cb32 = _mm(cb, R8)
    wgt = cb32 * jnp.exp(z32)
    den32 = wgt.sum(0, keepdims=True)
    den_pad = jnp.concatenate(
        [den32, jnp.zeros((1, 128 - MH), f32)], axis=1)
    res = lax.dot_general(wgt, hs_g, (((0,), (0,)), ((), ())),
                          preferred_element_type=f32)
    mnew_add = _mm(R8, res * M32)

    @pl.when(i == 0)
    def _():
        mnew_o[...] = jnp.zeros((NM, HID), f32)
        den_o[...] = jnp.zeros((NM, 128), f32)

    mnew_o[...] += mnew_add
    den_o[...] += jnp.broadcast_to(den_pad, (NM, 128))
    hs_m = _mm(hm0.astype(bf16), Wm2g[...])
    es_m = _mm(hs_m * asm2g[...], G64)
    wd = _mm(Wm2g[...].astype(f32) * adm2g[...], G64)
    ed_g = _mm(hg0, wd)
    z2 = _leaky(_flat_mh(es_m, T4, R8) + _mm(ed_g, T4))
    w2 = cb32 * jnp.exp(z2)
    al = w2 / (_mm(_mm(w2, T4T), T4) + 1e-16)
    hs2 = _mm(G4T, hs_m) * M32
    gnew = _mm(al, hs2) + bm2g[...]
    hg1 = jax.nn.gelu(_ln(gnew + hg0, lng0g[...], lng0b[...]))
    hgc_o[...] = jnp.concatenate(
        [hg1, cb, jnp.zeros((R, 128 - NM), f32)], axis=1)


def _tc_drugmlp_body(df, dW1, db1, l1g, l1b, dW2, db2, l2g, l2b, h2_o):
    h1 = jax.nn.gelu(_ln(_mm(df[...], dW1[...]) + db1[...], l1g[...], l1b[...]))
    h2_o[...] = jax.nn.gelu(_ln(_mm(h1, dW2[...]) + db2[...], l2g[...], l2b[...]))


def _tc_drugout_body(ablk, h2f, h2b, sWl, sbl, sWr, slng, slnb,
                     doW, dob, demb_o):
    A = ablk[...]
    deg = A.astype(jnp.float32).sum(-1, keepdims=True)
    msg = _mm(A, h2f[...]) / jnp.maximum(deg, 1.0)
    h = h2b[...]
    hn = _mm(msg, sWl[...]) + sbl[...] + _mm(h, sWr[...])
    hd = jax.nn.gelu(_ln(h + hn, slng[...], slnb[...]))
    demb_o[...] = _mm(hd, doW[...]) + dob[...]


def _tc_decode_body(H, C, geneh, de, mnew_un, den, mech, mW, mb, mlng,
                    mlnb, g2m0b, lnm0g, lnm0b, Wm2g1, asm2g1, adm2g1, bm2g1,
                    lng1g, lng1b, goW, gob, protos, gmW1, gmb1, gmW2v, gmb2v,
                    Wbil, sc_o):
    NM = mech.shape[0]
    HID = mW.shape[1]
    OUT = goW.shape[1]
    MH = NM * H
    f32 = jnp.float32
    G64 = _sel((HID, H), lambda r, c: (r // C) == c)
    T4 = _sel((H, MH), lambda r, c: (c % H) == r)
    T4T = _sel((MH, H), lambda r, c: (r % H) == c)
    R8 = _sel((NM, MH), lambda r, c: (c // H) == r)
    G4T = _sel((MH, NM), lambda r, c: (r // H) == c)
    M32 = _sel((MH, HID), lambda r, c: (r % H) == (c // C))
    hm0 = _ln(_mm(mech[...], mW[...]) + mb[...], mlng[...], mlnb[...])
    den_bc = jnp.broadcast_to(den[0:1, 0:MH], (NM, MH))
    dmat = _mm(R8 * den_bc, M32)
    mnew0 = mnew_un[...] / (dmat + 1e-16)
    hm1 = jax.nn.gelu(_ln(mnew0 + g2m0b[...] + hm0, lnm0g[...], lnm0b[...]))
    hs_m = _mm(hm1, Wm2g1[...])
    es_m = _mm(hs_m * asm2g1[...], G64)
    wd = _mm(Wm2g1[...] * adm2g1[...], G64)
    geh = geneh[:, 0:HID]
    cg = geneh[:, HID:HID + NM]
    ed_g = _mm(geh, wd)
    z = _leaky(_flat_mh(es_m, T4, R8) + _mm(ed_g, T4))
    w3 = _mm(cg, R8) * jnp.exp(z)
    al = w3 / (_mm(_mm(w3, T4T), T4) + 1e-16)
    hs2 = _mm(G4T, hs_m) * M32
    gnew = _mm(al, hs2) + bm2g1[...]
    hg2 = jax.nn.gelu(_ln(gnew + geh, lng1g[...], lng1b[...]))
    ge = _mm(hg2, goW[...]) + gob[...]
    u = _mm(ge, gmW1[0:OUT, :])
    v = _mm(protos[...], gmW1[OUT:2 * OUT, :])
    gact = jax.nn.gelu(u[:, None, :] + v[None, :, :] + gmb1[...][None])
    gates = (gact * gmW2v[...][None]).sum(-1) + gmb2v[...]
    mx = gates.max(-1, keepdims=True)
    ex = jnp.exp(gates - mx)
    w = ex / ex.sum(-1, keepdims=True)
    gfin = ge + _mm(w, protos[...])
    sc_o[...] = (_mm(gfin, Wbil[...]) * de[...]).sum(-1)


def kernel(gene_feat, mech_feat, drug_feat, params, gm_src, gm_dst,
           dd_edge_index, gene_idx, drug_idx):
    p = params
    NG, GFD = gene_feat.shape
    NM, MFD = mech_feat.shape
    ND, DFD = drug_feat.shape
    EGM = gm_src.shape[0]
    EDD = dd_edge_index.shape[1]
    B = gene_idx.shape[0]
    HID = p['gW'].shape[1]
    OUT = p['goW'].shape[1]
    H, C = p['g2m0_as'].shape
    f32 = jnp.float32
    bf16 = jnp.bfloat16

    RG = 512
    RD = 256
    P = 512
    NGP = _cdiv(NG, RG) * RG
    NDP = _cdiv(ND, 128) * 128
    CPAD = _cdiv(NG * NM + 1, NW * 8) * NW * 8
    CSL = CPAD // NW
    AROWS = NDP // (2 * NW)

    v2 = lambda a: a.reshape(1, -1)
    gf_p = jnp.pad(gene_feat, ((0, NGP - NG), (0, 0))).astype(bf16)
    df_p = jnp.pad(drug_feat, ((0, NDP - ND), (0, 0))).astype(bf16)

    kgm = _cdiv(EGM, NW * 128)
    egm_p = NW * kgm * 128
    gm_src2 = jnp.pad(gm_src, (0, egm_p - EGM),
                      constant_values=NG).reshape(-1, 128).astype(jnp.int32)
    gm_dst2 = jnp.pad(gm_dst, (0, egm_p - EGM)).reshape(-1, 128).astype(jnp.int32)

    kdd = _cdiv(EDD, NW * 128)
    edd_p = NW * kdd * 128
    dd_src2 = jnp.pad(dd_edge_index[0],
                      (0, edd_p - EDD)).reshape(-1, 128).astype(jnp.int32)
    dd_dst2 = jnp.pad(dd_edge_index[1], (0, edd_p - EDD),
                      constant_values=NDP - 1).reshape(-1, 128).astype(jnp.int32)
    CHG = 40
    CHD = 32

    kb = B // (NW * 128)
    gidx3 = gene_idx.reshape(NW, kb, 128).astype(jnp.int32)
    didx3 = drug_idx.reshape(NW, kb, 128).astype(jnp.int32)

    zc = jnp.zeros((CPAD,), f32)
    za = jnp.zeros((AROWS, NDP), f32)

    mesh = plsc.VectorSubcoreMesh(core_axis_name="c", subcore_axis_name="s",
                                  num_cores=NC, num_subcores=NS)

    sc_build = functools.partial(
        pl.kernel, mesh=mesh,
        compiler_params=pltpu.CompilerParams(needs_layout_passes=False,
                                             use_tc_tiling_on_sc=True),
        out_type=[jax.ShapeDtypeStruct((CPAD,), f32),
                  jax.ShapeDtypeStruct((NDP, NDP), f32)],
        scratch_types=[pltpu.VMEM((CHG, 128), jnp.int32),
                       pltpu.VMEM((CHG, 128), jnp.int32),
                       pltpu.VMEM((CHD, 128), jnp.int32),
                       pltpu.VMEM((CHD, 128), jnp.int32),
                       pltpu.VMEM((CSL,), f32),
                       pltpu.VMEM((AROWS, NDP), f32)],
    )(functools.partial(_sc_build, NM))
    c_flat, amat = sc_build(gm_src2, gm_dst2, dd_src2, dd_dst2, zc, za)

    cmat = jnp.pad(c_flat[:NG * NM].reshape(NG, NM), ((0, NGP - NG), (0, 0)))
    amat_bf = amat.astype(bf16)

    full = lambda shape: pl.BlockSpec(shape, lambda i: tuple(0 for _ in shape))
    h2 = pl.pallas_call(
        _tc_drugmlp_body,
        grid=(NDP // RD,),
        in_specs=[pl.BlockSpec((RD, DFD), lambda i: (i, 0)),
                  full((DFD, HID)), full((1, HID)), full((1, HID)), full((1, HID)),
                  full((HID, HID)), full((1, HID)), full((1, HID)), full((1, HID))],
        out_specs=pl.BlockSpec((RD, HID), lambda i: (i, 0)),
        out_shape=jax.ShapeDtypeStruct((NDP, HID), f32),
    )(df_p, p['dW1'].astype(bf16), v2(p['db1']), v2(p['dln1_g']), v2(p['dln1_b']),
      p['dW2'], v2(p['db2']), v2(p['dln2_g']), v2(p['dln2_b']))
    h2_bf = h2.astype(bf16)

    demb = pl.pallas_call(
        _tc_drugout_body,
        grid=(NDP // RD,),
        in_specs=[pl.BlockSpec((RD, NDP), lambda i: (i, 0)),
                  full((NDP, HID)),
                  pl.BlockSpec((RD, HID), lambda i: (i, 0)),
                  full((HID, HID)), full((1, HID)), full((HID, HID)),
                  full((1, HID)), full((1, HID)), full((HID, OUT)), full((1, OUT))],
        out_specs=pl.BlockSpec((RD, OUT), lambda i: (i, 0)),
        out_shape=jax.ShapeDtypeStruct((NDP, OUT), f32),
    )(amat_bf, h2_bf, h2,
      p['sWl'], v2(p['sbl']), p['sWr'], v2(p['sln_g']), v2(p['sln_b']),
      p['doW'], v2(p['dob']))

    hgc, mnew_un, den = pl.pallas_call(
        functools.partial(_tc_gene_body, H, C),
        grid=(NGP // RG,),
        in_specs=[pl.BlockSpec((RG, GFD), lambda i: (i, 0)),
                  pl.BlockSpec((RG, NM), lambda i: (i, 0)),
                  full((NM, MFD)),
                  full((GFD, HID)), full((1, HID)), full((1, HID)), full((1, HID)),
                  full((MFD, HID)), full((1, HID)), full((1, HID)), full((1, HID)),
                  full((HID, HID)), full((1, H * C)), full((1, H * C)),
                  full((HID, HID)), full((1, H * C)), full((1, H * C)),
                  full((1, HID)), full((1, HID)), full((1, HID))],
        out_specs=[pl.BlockSpec((RG, HID + 128), lambda i: (i, 0)),
                   full((NM, HID)), full((NM, 128))],
        out_shape=[jax.ShapeDtypeStruct((NGP, HID + 128), f32),
                   jax.ShapeDtypeStruct((NM, HID), f32),
                   jax.ShapeDtypeStruct((NM, 128), f32)],
    )(gf_p, cmat, mech_feat,
      p['gW'].astype(bf16), v2(p['gb']), v2(p['g_ln_g']), v2(p['g_ln_b']),
      p['mW'], v2(p['mb']), v2(p['m_ln_g']), v2(p['m_ln_b']),
      p['g2m0_W'].astype(bf16), v2(p['g2m0_as']), v2(p['g2m0_ad']),
      p['m2g0_W'].astype(bf16), v2(p['m2g0_as']), v2(p['m2g0_ad']),
      v2(p['m2g0_b']), v2(p['lng0_g']), v2(p['lng0_b']))

    sc_gather = functools.partial(
        pl.kernel, mesh=mesh,
        compiler_params=pltpu.CompilerParams(needs_layout_passes=False,
                                             use_tc_tiling_on_sc=True),
        out_type=[jax.ShapeDtypeStruct((B, HID + 128), f32),
                  jax.ShapeDtypeStruct((B, OUT), f32)],
        scratch_types=[pltpu.VMEM((kb, 128), jnp.int32),
                       pltpu.VMEM((kb, 128), jnp.int32),
                       pltpu.VMEM((128, HID + 128), f32),
                       pltpu.VMEM((128, OUT), f32),
                       pltpu.SemaphoreType.DMA],
    )(_sc_gather)
    geneh, de = sc_gather(hgc, demb, gidx3, didx3)

    scores = pl.pallas_call(
        functools.partial(_tc_decode_body, H, C),
        grid=(B // P,),
        in_specs=[pl.BlockSpec((P, HID + 128), lambda i: (i, 0)),
                  pl.BlockSpec((P, OUT), lambda i: (i, 0)),
                  full((NM, HID)), full((NM, 128)), full((NM, MFD)),
                  full((MFD, HID)), full((1, HID)), full((1, HID)), full((1, HID)),
                  full((1, HID)), full((1, HID)), full((1, HID)),
                  full((HID, HID)), full((1, H * C)), full((1, H * C)),
                  full((1, HID)), full((1, HID)), full((1, HID)),
                  full((HID, OUT)), full((1, OUT)), full((NM, OUT)),
                  full((2 * OUT, OUT)), full((1, OUT)), full((1, OUT)),
                  full((1, NM)), full((OUT, OUT))],
        out_specs=pl.BlockSpec((P,), lambda i: (i,)),
        out_shape=jax.ShapeDtypeStruct((B,), f32),
    )(geneh, de, mnew_un, den, mech_feat,
      p['mW'], v2(p['mb']), v2(p['m_ln_g']), v2(p['m_ln_b']),
      v2(p['g2m0_b']), v2(p['lnm0_g']), v2(p['lnm0_b']),
      p['m2g1_W'], v2(p['m2g1_as']), v2(p['m2g1_ad']), v2(p['m2g1_b']),
      v2(p['lng1_g']), v2(p['lng1_b']),
      p['goW'], v2(p['gob']), p['protos'],
      p['gmW1'], v2(p['gmb1']), p['gmW2'].reshape(1, OUT),
      jnp.broadcast_to(p['gmb2'].reshape(1, 1), (1, NM)), p['Wbil'])

    return scores


# speedup vs baseline: 45.9271x; 1.5032x over previous
import functools

import jax
import jax.numpy as jnp
from jax import lax
from jax.experimental import pallas as pl
from jax.experimental.pallas import tpu as pltpu
from jax.experimental.pallas import tpu_sc as plsc

NC = 2
NS = 16
NW = NC * NS
LEAK = 0.2


def _cdiv(a, b):
    return (a + b - 1) // b


def _ln(x, g, b):
    m = x.mean(-1, keepdims=True)
    v = ((x - m) ** 2).mean(-1, keepdims=True)
    return (x - m) / jnp.sqrt(v + 1e-5) * g + b


def _leaky(x):
    return jnp.where(x >= 0, x, LEAK * x)


def _mm(a, b):
    return jnp.dot(a, b, preferred_element_type=jnp.float32)


def _iota2(shape, d):
    return lax.broadcasted_iota(jnp.int32, shape, d)


def _sel(shape, fn):
    return fn(_iota2(shape, 0), _iota2(shape, 1)).astype(jnp.float32)


def _flat_mh(a, T4, R8):
    return (R8 * jnp.dot(a, T4, preferred_element_type=jnp.float32)
            ).sum(0, keepdims=True)


def _sc_build(nm, gm_src2, gm_dst2, dd_src2, dd_dst2, zc, za,
              c_out, a_out,
              src_v, dst_v, dsrc_v, ddst_v, acc_c, acc_a):
    rg = gm_src2.shape[0]
    chg = src_v.shape[0]
    rd = dd_src2.shape[0]
    chd = dsrc_v.shape[0]
    csl = acc_c.shape[0]
    arows = acc_a.shape[0]
    cid = lax.axis_index("c")
    sid = lax.axis_index("s")
    wid = sid * NC + cid
    ones16 = jnp.full((16,), 1.0, jnp.float32)

    lo_c = wid * csl
    pltpu.sync_copy(zc.at[pl.ds(pl.multiple_of(lo_c, 8), csl)], acc_c)
    for t in range(rg // chg):
        pltpu.sync_copy(gm_src2.at[pl.ds(t * chg, chg)], src_v)
        pltpu.sync_copy(gm_dst2.at[pl.ds(t * chg, chg)], dst_v)

        def row(r, c2):
            for j in range(128 // 16):
                s = src_v[r, pl.ds(j * 16, 16)]
                d = dst_v[r, pl.ds(j * 16, 16)]
                loc = s * nm + d - lo_c
                msk = (loc >= 0) & (loc < csl)
                locc = jnp.clip(loc, 0, csl - 1)
                plsc.addupdate_scatter(acc_c, [locc], ones16, mask=msk)
            return c2

        lax.fori_loop(0, chg, row, 0)
    pltpu.sync_copy(acc_c, c_out.at[pl.ds(pl.multiple_of(lo_c, 8), csl)])

    for p in range(2):
        lo_r = wid * (2 * arows) + p * arows
        pltpu.sync_copy(za, acc_a)
        for t in range(rd // chd):
            pltpu.sync_copy(dd_src2.at[pl.ds(t * chd, chd)], dsrc_v)
            pltpu.sync_copy(dd_dst2.at[pl.ds(t * chd, chd)], ddst_v)

            def row2(r, c2):
                for j in range(128 // 16):
                    s = dsrc_v[r, pl.ds(j * 16, 16)]
                    d = ddst_v[r, pl.ds(j * 16, 16)]
                    rr = d - lo_r
                    msk = (rr >= 0) & (rr < arows)
                    rrc = jnp.clip(rr, 0, arows - 1)
                    plsc.addupdate_scatter(acc_a, [rrc, s], ones16, mask=msk)
                return c2

            lax.fori_loop(0, chd, row2, 0)
        pltpu.sync_copy(acc_a, a_out.at[pl.ds(pl.multiple_of(lo_r, 8), arows)])


def _sc_gather(hgc, demb, gidx3, didx3,
               geneh_out, de_out,
               gidx_v, didx_v, rows_h, rows_d, sem):
    kb = gidx_v.shape[0]
    cid = lax.axis_index("c")
    sid = lax.axis_index("s")
    wid = sid * NC + cid
    pltpu.sync_copy(gidx3.at[wid], gidx_v)
    pltpu.sync_copy(didx3.at[wid], didx_v)
    for j in range(kb):
        base = pl.multiple_of(wid * (kb * 128) + j * 128, 8)
        pltpu.async_copy(hgc.at[gidx_v.at[j]], rows_h, sem).wait()
        pltpu.sync_copy(rows_h, geneh_out.at[pl.ds(base, 128)])
        pltpu.async_copy(demb.at[didx_v.at[j]], rows_d, sem).wait()
        pltpu.sync_copy(rows_d, de_out.at[pl.ds(base, 128)])


def _tc_gene_body(H, C, NG, gf, cm, mech, gW, gb, glng, glnb, mW, mb, mlng,
                  mlnb, Wg2m, asg2m, adg2m, Wm2g, asm2g, adm2g, bm2g, lng0g,
                  lng0b, hgc_o, mnew_o, den_o):
    i = pl.program_id(0)
    NM = mech.shape[0]
    HID = gW.shape[1]
    R = gf.shape[0]
    MH = NM * H
    bf16 = jnp.bfloat16
    f32 = jnp.float32
    x = gf[...].astype(bf16)
    rowok = ((i * R + lax.broadcasted_iota(jnp.int32, (R, 1), 0)) < NG
             ).astype(f32)
    hg0 = _ln(_mm(x, gW[...]) + gb[...], glng[...], glnb[...])
    hm0 = _ln(_mm(mech[...], mW[...]) + mb[...], mlng[...], mlnb[...])
    cb = cm[...]
    G64 = _sel((HID, H), lambda r, c: (r // C) == c)
    T4 = _sel((H, MH), lambda r, c: (c % H) == r)
    T4T = _sel((MH, H), lambda r, c: (r % H) == c)
    R8 = _sel((NM, MH), lambda r, c: (c // H) == r)
    G4T = _sel((MH, NM), lambda r, c: (r // H) == c)
    M32 = _sel((MH, HID), lambda r, c: (r % H) == (c // C))
    hg0b = hg0.astype(bf16)
    hs_g = _mm(hg0b, Wg2m[...])
    es_g = _mm(hs_g * asg2m[...], G64)
    ed_m = _mm(_mm(hm0.astype(bf16), Wg2m[...]) * adg2m[...], G64)
    z32 = _leaky(_mm(es_g, T4) + _flat_mh(ed_m, T4, R8))
    cb32 = _mm(cb, R8)
    wgt = jnp.where(rowok > 0, cb32 * jnp.exp(jnp.minimum(z32, 60.0)), 0.0)
    den32 = wgt.sum(0, keepdims=True)
    den_pad = jnp.concatenate(
        [den32, jnp.zeros((1, 128 - MH), f32)], axis=1)
    res = lax.dot_general(wgt, hs_g, (((0,), (0,)), ((), ())),
                          preferred_element_type=f32)
    mnew_add = _mm(R8, res * M32)

    @pl.when(i == 0)
    def _():
        mnew_o[...] = jnp.zeros((NM, HID), f32)
        den_o[...] = jnp.zeros((NM, 128), f32)

    mnew_o[...] += mnew_add
    den_o[...] += jnp.broadcast_to(den_pad, (NM, 128))
    hs_m = _mm(hm0.astype(bf16), Wm2g[...])
    es_m = _mm(hs_m * asm2g[...], G64)
    wd = _mm(Wm2g[...].astype(f32) * adm2g[...], G64)
    ed_g = _mm(hg0, wd)
    z2 = _leaky(_flat_mh(es_m, T4, R8) + _mm(ed_g, T4))
    w2 = cb32 * jnp.exp(z2)
    al = w2 / (_mm(_mm(w2, T4T), T4) + 1e-16)
    hs2 = _mm(G4T, hs_m) * M32
    gnew = _mm(al, hs2) + bm2g[...]
    hg1 = jax.nn.gelu(_ln(gnew + hg0, lng0g[...], lng0b[...]))
    hgc_o[...] = jnp.concatenate(
        [hg1, cb, jnp.zeros((R, 128 - NM), f32)], axis=1)


def _tc_drugmlp_body(df, dW1, db1, l1g, l1b, dW2, db2, l2g, l2b, h2_o):
    x = df[...].astype(jnp.bfloat16)
    h1 = jax.nn.gelu(_ln(_mm(x, dW1[...]) + db1[...], l1g[...], l1b[...]))
    h2_o[...] = jax.nn.gelu(_ln(_mm(h1.astype(jnp.bfloat16), dW2[...])
                                + db2[...], l2g[...], l2b[...]))


def _tc_drugout_body(ablk, h2f, h2b, sWl, sbl, sWr, slng, slnb,
                     doW, dob, demb_o):
    A = ablk[...]
    deg = A.sum(-1, keepdims=True)
    msg = (_mm(A.astype(jnp.bfloat16), h2f[...].astype(jnp.bfloat16))
           / jnp.maximum(deg, 1.0))
    h = h2b[...]
    hn = _mm(msg, sWl[...]) + sbl[...] + _mm(h, sWr[...])
    hd = jax.nn.gelu(_ln(h + hn, slng[...], slnb[...]))
    demb_o[...] = _mm(hd, doW[...]) + dob[...]


def _tc_decode_body(H, C, geneh, de, mnew_un, den, mech, mW, mb, mlng,
                    mlnb, g2m0b, lnm0g, lnm0b, Wm2g1, asm2g1, adm2g1, bm2g1,
                    lng1g, lng1b, goW, gob, protos, gmW1, gmb1, gmW2v, gmb2v,
                    Wbil, sc_o):
    NM = mech.shape[0]
    HID = mW.shape[1]
    OUT = goW.shape[1]
    MH = NM * H
    f32 = jnp.float32
    G64 = _sel((HID, H), lambda r, c: (r // C) == c)
    T4 = _sel((H, MH), lambda r, c: (c % H) == r)
    T4T = _sel((MH, H), lambda r, c: (r % H) == c)
    R8 = _sel((NM, MH), lambda r, c: (c // H) == r)
    G4T = _sel((MH, NM), lambda r, c: (r // H) == c)
    M32 = _sel((MH, HID), lambda r, c: (r % H) == (c // C))
    hm0 = _ln(_mm(mech[...], mW[...]) + mb[...], mlng[...], mlnb[...])
    den_bc = jnp.broadcast_to(den[0:1, 0:MH], (NM, MH))
    dmat = _mm(R8 * den_bc, M32)
    mnew0 = mnew_un[...] / (dmat + 1e-16)
    hm1 = jax.nn.gelu(_ln(mnew0 + g2m0b[...] + hm0, lnm0g[...], lnm0b[...]))
    hs_m = _mm(hm1, Wm2g1[...])
    es_m = _mm(hs_m * asm2g1[...], G64)
    wd = _mm(Wm2g1[...] * adm2g1[...], G64)
    geh = geneh[:, 0:HID]
    cg = geneh[:, HID:HID + NM]
    ed_g = _mm(geh, wd)
    z = _leaky(_flat_mh(es_m, T4, R8) + _mm(ed_g, T4))
    w3 = _mm(cg, R8) * jnp.exp(z)
    al = w3 / (_mm(_mm(w3, T4T), T4) + 1e-16)
    hs2 = _mm(G4T, hs_m) * M32
    gnew = _mm(al, hs2) + bm2g1[...]
    hg2 = jax.nn.gelu(_ln(gnew + geh, lng1g[...], lng1b[...]))
    ge = _mm(hg2, goW[...]) + gob[...]
    u = _mm(ge, gmW1[0:OUT, :])
    v = _mm(protos[...], gmW1[OUT:2 * OUT, :])
    gact = jax.nn.gelu(u[:, None, :] + v[None, :, :] + gmb1[...][None])
    gates = (gact * gmW2v[...][None]).sum(-1) + gmb2v[...]
    mx = gates.max(-1, keepdims=True)
    ex = jnp.exp(gates - mx)
    w = ex / ex.sum(-1, keepdims=True)
    gfin = ge + _mm(w, protos[...])
    sc_o[...] = (_mm(gfin, Wbil[...]) * de[...]).sum(-1)


def kernel(gene_feat, mech_feat, drug_feat, params, gm_src, gm_dst,
           dd_edge_index, gene_idx, drug_idx):
    p = params
    NG, GFD = gene_feat.shape
    NM, MFD = mech_feat.shape
    ND, DFD = drug_feat.shape
    EGM = gm_src.shape[0]
    EDD = dd_edge_index.shape[1]
    B = gene_idx.shape[0]
    HID = p['gW'].shape[1]
    OUT = p['goW'].shape[1]
    H, C = p['g2m0_as'].shape
    f32 = jnp.float32
    bf16 = jnp.bfloat16

    RG = 1024
    RD = 256
    P = 512
    NDP = _cdiv(ND, 128) * 128
    CPAD = _cdiv(NG * NM + 1, NW * 8) * NW * 8
    CSL = CPAD // NW
    AROWS = NDP // (2 * NW)

    v2 = lambda a: a.reshape(1, -1)

    kgm = _cdiv(EGM, NW * 128)
    egm_p = NW * kgm * 128
    gm_src2 = jnp.pad(gm_src, (0, egm_p - EGM),
                      constant_values=NG).reshape(-1, 128).astype(jnp.int32)
    gm_dst2 = jnp.pad(gm_dst, (0, egm_p - EGM)).reshape(-1, 128).astype(jnp.int32)

    kdd = _cdiv(EDD, NW * 128)
    edd_p = NW * kdd * 128
    dd_src2 = jnp.pad(dd_edge_index[0],
                      (0, edd_p - EDD)).reshape(-1, 128).astype(jnp.int32)
    dd_dst2 = jnp.pad(dd_edge_index[1], (0, edd_p - EDD),
                      constant_values=NDP - 1).reshape(-1, 128).astype(jnp.int32)
    CHG = 80
    CHD = 64

    kb = B // (NW * 128)
    gidx3 = gene_idx.reshape(NW, kb, 128).astype(jnp.int32)
    didx3 = drug_idx.reshape(NW, kb, 128).astype(jnp.int32)

    zc = jnp.zeros((CPAD,), f32)
    za = jnp.zeros((AROWS, NDP), f32)

    mesh = plsc.VectorSubcoreMesh(core_axis_name="c", subcore_axis_name="s",
                                  num_cores=NC, num_subcores=NS)

    sc_build = functools.partial(
        pl.kernel, mesh=mesh,
        compiler_params=pltpu.CompilerParams(needs_layout_passes=False,
                                             use_tc_tiling_on_sc=True),
        out_type=[jax.ShapeDtypeStruct((CPAD,), f32),
                  jax.ShapeDtypeStruct((NDP, NDP), f32)],
        scratch_types=[pltpu.VMEM((CHG, 128), jnp.int32),
                       pltpu.VMEM((CHG, 128), jnp.int32),
                       pltpu.VMEM((CHD, 128), jnp.int32),
                       pltpu.VMEM((CHD, 128), jnp.int32),
                       pltpu.VMEM((CSL,), f32),
                       pltpu.VMEM((AROWS, NDP), f32)],
    )(functools.partial(_sc_build, NM))
    c_flat, amat = sc_build(gm_src2, gm_dst2, dd_src2, dd_dst2, zc, za)

    cmat = c_flat[:NG * NM].reshape(NG, NM)

    full = lambda shape: pl.BlockSpec(shape, lambda i: tuple(0 for _ in shape))
    h2 = pl.pallas_call(
        _tc_drugmlp_body,
        grid=(_cdiv(ND, RD),),
        in_specs=[pl.BlockSpec((RD, DFD), lambda i: (i, 0)),
                  full((DFD, HID)), full((1, HID)), full((1, HID)), full((1, HID)),
                  full((HID, HID)), full((1, HID)), full((1, HID)), full((1, HID))],
        out_specs=pl.BlockSpec((RD, HID), lambda i: (i, 0)),
        out_shape=jax.ShapeDtypeStruct((ND, HID), f32),
    )(drug_feat, p['dW1'].astype(bf16), v2(p['db1']), v2(p['dln1_g']),
      v2(p['dln1_b']), p['dW2'].astype(bf16), v2(p['db2']), v2(p['dln2_g']),
      v2(p['dln2_b']))
    h2p = jnp.pad(h2, ((0, NDP - ND), (0, 0)))

    demb = pl.pallas_call(
        _tc_drugout_body,
        grid=(_cdiv(ND, RD),),
        in_specs=[pl.BlockSpec((RD, NDP), lambda i: (i, 0)),
                  full((NDP, HID)),
                  pl.BlockSpec((RD, HID), lambda i: (i, 0)),
                  full((HID, HID)), full((1, HID)), full((HID, HID)),
                  full((1, HID)), full((1, HID)), full((HID, OUT)), full((1, OUT))],
        out_specs=pl.BlockSpec((RD, OUT), lambda i: (i, 0)),
        out_shape=jax.ShapeDtypeStruct((ND, OUT), f32),
    )(amat, h2p, h2p,
      p['sWl'], v2(p['sbl']), p['sWr'], v2(p['sln_g']), v2(p['sln_b']),
      p['doW'], v2(p['dob']))

    hgc, mnew_un, den = pl.pallas_call(
        functools.partial(_tc_gene_body, H, C, NG),
        grid=(_cdiv(NG, RG),),
        in_specs=[pl.BlockSpec((RG, GFD), lambda i: (i, 0)),
                  pl.BlockSpec((RG, NM), lambda i: (i, 0)),
                  full((NM, MFD)),
                  full((GFD, HID)), full((1, HID)), full((1, HID)), full((1, HID)),
                  full((MFD, HID)), full((1, HID)), full((1, HID)), full((1, HID)),
                  full((HID, HID)), full((1, H * C)), full((1, H * C)),
                  full((HID, HID)), full((1, H * C)), full((1, H * C)),
                  full((1, HID)), full((1, HID)), full((1, HID))],
        out_specs=[pl.BlockSpec((RG, HID + 128), lambda i: (i, 0)),
                   full((NM, HID)), full((NM, 128))],
        out_shape=[jax.ShapeDtypeStruct((NG, HID + 128), f32),
                   jax.ShapeDtypeStruct((NM, HID), f32),
                   jax.ShapeDtypeStruct((NM, 128), f32)],
    )(gene_feat, cmat, mech_feat,
      p['gW'].astype(bf16), v2(p['gb']), v2(p['g_ln_g']), v2(p['g_ln_b']),
      p['mW'], v2(p['mb']), v2(p['m_ln_g']), v2(p['m_ln_b']),
      p['g2m0_W'].astype(bf16), v2(p['g2m0_as']), v2(p['g2m0_ad']),
      p['m2g0_W'].astype(bf16), v2(p['m2g0_as']), v2(p['m2g0_ad']),
      v2(p['m2g0_b']), v2(p['lng0_g']), v2(p['lng0_b']))

    sc_gather = functools.partial(
        pl.kernel, mesh=mesh,
        compiler_params=pltpu.CompilerParams(needs_layout_passes=False,
                                             use_tc_tiling_on_sc=True),
        out_type=[jax.ShapeDtypeStruct((B, HID + 128), f32),
                  jax.ShapeDtypeStruct((B, OUT), f32)],
        scratch_types=[pltpu.VMEM((kb, 128), jnp.int32),
                       pltpu.VMEM((kb, 128), jnp.int32),
                       pltpu.VMEM((128, HID + 128), f32),
                       pltpu.VMEM((128, OUT), f32),
                       pltpu.SemaphoreType.DMA],
    )(_sc_gather)
    geneh, de = sc_gather(hgc, demb, gidx3, didx3)

    scores = pl.pallas_call(
        functools.partial(_tc_decode_body, H, C),
        grid=(B // P,),
        in_specs=[pl.BlockSpec((P, HID + 128), lambda i: (i, 0)),
                  pl.BlockSpec((P, OUT), lambda i: (i, 0)),
                  full((NM, HID)), full((NM, 128)), full((NM, MFD)),
                  full((MFD, HID)), full((1, HID)), full((1, HID)), full((1, HID)),
                  full((1, HID)), full((1, HID)), full((1, HID)),
                  full((HID, HID)), full((1, H * C)), full((1, H * C)),
                  full((1, HID)), full((1, HID)), full((1, HID)),
                  full((HID, OUT)), full((1, OUT)), full((NM, OUT)),
                  full((2 * OUT, OUT)), full((1, OUT)), full((1, OUT)),
                  full((1, NM)), full((OUT, OUT))],
        out_specs=pl.BlockSpec((P,), lambda i: (i,)),
        out_shape=jax.ShapeDtypeStruct((B,), f32),
    )(geneh, de, mnew_un, den, mech_feat,
      p['mW'], v2(p['mb']), v2(p['m_ln_g']), v2(p['m_ln_b']),
      v2(p['g2m0_b']), v2(p['lnm0_g']), v2(p['lnm0_b']),
      p['m2g1_W'], v2(p['m2g1_as']), v2(p['m2g1_ad']), v2(p['m2g1_b']),
      v2(p['lng1_g']), v2(p['lng1_b']),
      p['goW'], v2(p['gob']), p['protos'],
      p['gmW1'], v2(p['gmb1']), p['gmW2'].reshape(1, OUT),
      jnp.broadcast_to(p['gmb2'].reshape(1, 1), (1, NM)), p['Wbil'])

    return scores


# speedup vs baseline: 46.5291x; 1.0131x over previous
import functools

import jax
import jax.numpy as jnp
from jax import lax
from jax.experimental import pallas as pl
from jax.experimental.pallas import tpu as pltpu
from jax.experimental.pallas import tpu_sc as plsc

NC = 2
NS = 16
NW = NC * NS
LEAK = 0.2


def _cdiv(a, b):
    return (a + b - 1) // b


def _ln(x, g, b):
    m = x.mean(-1, keepdims=True)
    v = ((x - m) ** 2).mean(-1, keepdims=True)
    return (x - m) / jnp.sqrt(v + 1e-5) * g + b


def _leaky(x):
    return jnp.where(x >= 0, x, LEAK * x)


def _mm(a, b):
    return jnp.dot(a, b, preferred_element_type=jnp.float32)


def _iota2(shape, d):
    return lax.broadcasted_iota(jnp.int32, shape, d)


def _sel(shape, fn):
    return fn(_iota2(shape, 0), _iota2(shape, 1)).astype(jnp.float32)


def _flat_mh(a, T4, R8):
    return (R8 * jnp.dot(a, T4, preferred_element_type=jnp.float32)
            ).sum(0, keepdims=True)


def _sc_counts(nm, gm_src2, gm_dst2, zc,
               c_out,
               src_v, dst_v, acc_c):
    rg = gm_src2.shape[0]
    chg = src_v.shape[0]
    csl = acc_c.shape[0]
    cid = lax.axis_index("c")
    sid = lax.axis_index("s")
    wid = sid * NC + cid
    ones16 = jnp.full((16,), 1.0, jnp.float32)
    lo_c = wid * csl
    pltpu.sync_copy(zc.at[pl.ds(pl.multiple_of(lo_c, 8), csl)], acc_c)
    for t in range(rg // chg):
        pltpu.sync_copy(gm_src2.at[pl.ds(t * chg, chg)], src_v)
        pltpu.sync_copy(gm_dst2.at[pl.ds(t * chg, chg)], dst_v)

        def row(r, c2):
            for j in range(128 // 16):
                s = src_v[r, pl.ds(j * 16, 16)]
                d = dst_v[r, pl.ds(j * 16, 16)]
                loc = s * nm + d - lo_c
                msk = (loc >= 0) & (loc < csl)
                locc = jnp.clip(loc, 0, csl - 1)
                plsc.addupdate_scatter(acc_c, [locc], ones16, mask=msk)
            return c2

        lax.fori_loop(0, chg, row, 0)
    pltpu.sync_copy(acc_c, c_out.at[pl.ds(pl.multiple_of(lo_c, 8), csl)])


def _sc_adj(dd_src2, dd_dst2, za,
            a_out,
            dsrc_v, ddst_v, acc_a):
    rd = dd_src2.shape[0]
    chd = dsrc_v.shape[0]
    arows = acc_a.shape[0]
    cid = lax.axis_index("c")
    sid = lax.axis_index("s")
    wid = sid * NC + cid
    ones16 = jnp.full((16,), 1.0, jnp.float32)
    for p in range(2):
        lo_r = wid * (2 * arows) + p * arows
        pltpu.sync_copy(za, acc_a)
        for t in range(rd // chd):
            pltpu.sync_copy(dd_src2.at[pl.ds(t * chd, chd)], dsrc_v)
            pltpu.sync_copy(dd_dst2.at[pl.ds(t * chd, chd)], ddst_v)

            def row2(r, c2):
                for j in range(128 // 16):
                    s = dsrc_v[r, pl.ds(j * 16, 16)]
                    d = ddst_v[r, pl.ds(j * 16, 16)]
                    rr = d - lo_r
                    msk = (rr >= 0) & (rr < arows)
                    rrc = jnp.clip(rr, 0, arows - 1)
                    plsc.addupdate_scatter(acc_a, [rrc, s], ones16, mask=msk)
                return c2

            lax.fori_loop(0, chd, row2, 0)
        pltpu.sync_copy(acc_a, a_out.at[pl.ds(pl.multiple_of(lo_r, 8), arows)])


def _sc_gather(hgc, demb, gidx3, didx3,
               geneh_out, de_out,
               gidx_v, didx_v, rows_h, rows_d, sem):
    kb = gidx_v.shape[0]
    cid = lax.axis_index("c")
    sid = lax.axis_index("s")
    wid = sid * NC + cid
    pltpu.sync_copy(gidx3.at[wid], gidx_v)
    pltpu.sync_copy(didx3.at[wid], didx_v)
    for j in range(kb):
        base = pl.multiple_of(wid * (kb * 128) + j * 128, 8)
        pltpu.async_copy(hgc.at[gidx_v.at[j]], rows_h, sem).wait()
        pltpu.sync_copy(rows_h, geneh_out.at[pl.ds(base, 128)])
        pltpu.async_copy(demb.at[didx_v.at[j]], rows_d, sem).wait()
        pltpu.sync_copy(rows_d, de_out.at[pl.ds(base, 128)])


def _tc_gene_body(H, C, NG, gf, cm, mech, gW, gb, glng, glnb, mW, mb, mlng,
                  mlnb, Wg2m, asg2m, adg2m, Wm2g, asm2g, adm2g, bm2g, lng0g,
                  lng0b, hgc_o, mnew_o, den_o):
    i = pl.program_id(0)
    NM = mech.shape[0]
    HID = gW.shape[1]
    R = gf.shape[0]
    MH = NM * H
    bf16 = jnp.bfloat16
    f32 = jnp.float32
    x = gf[...].astype(bf16)
    rowok = ((i * R + lax.broadcasted_iota(jnp.int32, (R, 1), 0)) < NG
             ).astype(f32)
    hg0 = _ln(_mm(x, gW[...]) + gb[...], glng[...], glnb[...])
    hm0 = _ln(_mm(mech[...], mW[...]) + mb[...], mlng[...], mlnb[...])
    cb = cm[...]
    G64 = _sel((HID, H), lambda r, c: (r // C) == c)
    T4 = _sel((H, MH), lambda r, c: (c % H) == r)
    T4T = _sel((MH, H), lambda r, c: (r % H) == c)
    R8 = _sel((NM, MH), lambda r, c: (c // H) == r)
    G4T = _sel((MH, NM), lambda r, c: (r // H) == c)
    M32 = _sel((MH, HID), lambda r, c: (r % H) == (c // C))
    hg0b = hg0.astype(bf16)
    hs_g = _mm(hg0b, Wg2m[...])
    es_g = _mm(hs_g * asg2m[...], G64)
    ed_m = _mm(_mm(hm0.astype(bf16), Wg2m[...]) * adg2m[...], G64)
    z32 = _leaky(_mm(es_g, T4) + _flat_mh(ed_m, T4, R8))
    cb32 = _mm(cb, R8)
    wgt = jnp.where(rowok > 0, cb32 * jnp.exp(jnp.minimum(z32, 60.0)), 0.0)
    den32 = wgt.sum(0, keepdims=True)
    den_pad = jnp.concatenate(
        [den32, jnp.zeros((1, 128 - MH), f32)], axis=1)
    res = lax.dot_general(wgt, hs_g, (((0,), (0,)), ((), ())),
                          preferred_element_type=f32)
    mnew_add = _mm(R8, res * M32)

    @pl.when(i == 0)
    def _():
        mnew_o[...] = jnp.zeros((NM, HID), f32)
        den_o[...] = jnp.zeros((NM, 128), f32)

    mnew_o[...] += mnew_add
    den_o[...] += jnp.broadcast_to(den_pad, (NM, 128))
    hs_m = _mm(hm0.astype(bf16), Wm2g[...])
    es_m = _mm(hs_m * asm2g[...], G64)
    wd = _mm(Wm2g[...].astype(f32) * adm2g[...], G64)
    ed_g = _mm(hg0, wd)
    z2 = _leaky(_flat_mh(es_m, T4, R8) + _mm(ed_g, T4))
    w2 = cb32 * jnp.exp(z2)
    al = w2 / (_mm(_mm(w2, T4T), T4) + 1e-16)
    hs2 = _mm(G4T, hs_m) * M32
    gnew = _mm(al, hs2) + bm2g[...]
    hg1 = jax.nn.gelu(_ln(gnew + hg0, lng0g[...], lng0b[...]))
    hgc_o[...] = jnp.concatenate(
        [hg1, cb, jnp.zeros((R, 128 - NM), f32)], axis=1)


def _tc_drugmlp_body(df, dW1, db1, l1g, l1b, dW2, db2, l2g, l2b, h2_o):
    x = df[...].astype(jnp.bfloat16)
    h1 = jax.nn.gelu(_ln(_mm(x, dW1[...]) + db1[...], l1g[...], l1b[...]))
    h2_o[...] = jax.nn.gelu(_ln(_mm(h1.astype(jnp.bfloat16), dW2[...])
                                + db2[...], l2g[...], l2b[...]))


def _tc_drugout_body(ablk, h2f, h2b, sWl, sbl, sWr, slng, slnb,
                     doW, dob, demb_o):
    A = ablk[...]
    deg = A.sum(-1, keepdims=True)
    msg = (_mm(A.astype(jnp.bfloat16), h2f[...].astype(jnp.bfloat16))
           / jnp.maximum(deg, 1.0))
    h = h2b[...]
    hn = _mm(msg, sWl[...]) + sbl[...] + _mm(h, sWr[...])
    hd = jax.nn.gelu(_ln(h + hn, slng[...], slnb[...]))
    demb_o[...] = _mm(hd, doW[...]) + dob[...]


def _tc_decode_body(H, C, geneh, de, mnew_un, den, mech, mW, mb, mlng,
                    mlnb, g2m0b, lnm0g, lnm0b, Wm2g1, asm2g1, adm2g1, bm2g1,
                    lng1g, lng1b, goW, gob, protos, gmW1, gmb1, gmW2v, gmb2v,
                    Wbil, sc_o):
    NM = mech.shape[0]
    HID = mW.shape[1]
    OUT = goW.shape[1]
    MH = NM * H
    f32 = jnp.float32
    G64 = _sel((HID, H), lambda r, c: (r // C) == c)
    T4 = _sel((H, MH), lambda r, c: (c % H) == r)
    T4T = _sel((MH, H), lambda r, c: (r % H) == c)
    R8 = _sel((NM, MH), lambda r, c: (c // H) == r)
    G4T = _sel((MH, NM), lambda r, c: (r // H) == c)
    M32 = _sel((MH, HID), lambda r, c: (r % H) == (c // C))
    hm0 = _ln(_mm(mech[...], mW[...]) + mb[...], mlng[...], mlnb[...])
    den_bc = jnp.broadcast_to(den[0:1, 0:MH], (NM, MH))
    dmat = _mm(R8 * den_bc, M32)
    mnew0 = mnew_un[...] / (dmat + 1e-16)
    hm1 = jax.nn.gelu(_ln(mnew0 + g2m0b[...] + hm0, lnm0g[...], lnm0b[...]))
    hs_m = _mm(hm1, Wm2g1[...])
    es_m = _mm(hs_m * asm2g1[...], G64)
    wd = _mm(Wm2g1[...] * adm2g1[...], G64)
    geh = geneh[:, 0:HID]
    cg = geneh[:, HID:HID + NM]
    ed_g = _mm(geh, wd)
    z = _leaky(_flat_mh(es_m, T4, R8) + _mm(ed_g, T4))
    w3 = _mm(cg, R8) * jnp.exp(z)
    al = w3 / (_mm(_mm(w3, T4T), T4) + 1e-16)
    hs2 = _mm(G4T, hs_m) * M32
    gnew = _mm(al, hs2) + bm2g1[...]
    hg2 = jax.nn.gelu(_ln(gnew + geh, lng1g[...], lng1b[...]))
    ge = _mm(hg2, goW[...]) + gob[...]
    u = _mm(ge, gmW1[0:OUT, :])
    v = _mm(protos[...], gmW1[OUT:2 * OUT, :])
    gin = (u[:, None, :] + v[None, :, :] + gmb1[...][None]).astype(jnp.bfloat16)
    gact = jax.nn.gelu(gin)
    gates = ((gact * gmW2v[...][None].astype(jnp.bfloat16))
             .astype(f32).sum(-1) + gmb2v[...])
    mx = gates.max(-1, keepdims=True)
    ex = jnp.exp(gates - mx)
    w = ex / ex.sum(-1, keepdims=True)
    gfin = ge + _mm(w, protos[...])
    sc_o[...] = (_mm(gfin, Wbil[...]) * de[...]).sum(-1)


def kernel(gene_feat, mech_feat, drug_feat, params, gm_src, gm_dst,
           dd_edge_index, gene_idx, drug_idx):
    p = params
    NG, GFD = gene_feat.shape
    NM, MFD = mech_feat.shape
    ND, DFD = drug_feat.shape
    EGM = gm_src.shape[0]
    EDD = dd_edge_index.shape[1]
    B = gene_idx.shape[0]
    HID = p['gW'].shape[1]
    OUT = p['goW'].shape[1]
    H, C = p['g2m0_as'].shape
    f32 = jnp.float32
    bf16 = jnp.bfloat16

    RG = 1024
    RD = 256
    P = 512
    NDP = _cdiv(ND, 128) * 128
    CPAD = _cdiv(NG * NM + 1, NW * 8) * NW * 8
    CSL = CPAD // NW
    AROWS = NDP // (2 * NW)

    v2 = lambda a: a.reshape(1, -1)

    kgm = _cdiv(EGM, NW * 128)
    egm_p = NW * kgm * 128
    gm_src2 = jnp.pad(gm_src, (0, egm_p - EGM),
                      constant_values=NG).reshape(-1, 128).astype(jnp.int32)
    gm_dst2 = jnp.pad(gm_dst, (0, egm_p - EGM)).reshape(-1, 128).astype(jnp.int32)

    kdd = _cdiv(EDD, NW * 128)
    edd_p = NW * kdd * 128
    dd_src2 = jnp.pad(dd_edge_index[0],
                      (0, edd_p - EDD)).reshape(-1, 128).astype(jnp.int32)
    dd_dst2 = jnp.pad(dd_edge_index[1], (0, edd_p - EDD),
                      constant_values=NDP - 1).reshape(-1, 128).astype(jnp.int32)
    CHG = 80
    CHD = 64

    kb = B // (NW * 128)
    gidx3 = gene_idx.reshape(NW, kb, 128).astype(jnp.int32)
    didx3 = drug_idx.reshape(NW, kb, 128).astype(jnp.int32)

    zc = jnp.zeros((CPAD,), f32)
    za = jnp.zeros((AROWS, NDP), f32)

    mesh = plsc.VectorSubcoreMesh(core_axis_name="c", subcore_axis_name="s",
                                  num_cores=NC, num_subcores=NS)

    sc_counts = functools.partial(
        pl.kernel, mesh=mesh,
        compiler_params=pltpu.CompilerParams(needs_layout_passes=False,
                                             use_tc_tiling_on_sc=True),
        out_type=[jax.ShapeDtypeStruct((CPAD,), f32)],
        scratch_types=[pltpu.VMEM((CHG, 128), jnp.int32),
                       pltpu.VMEM((CHG, 128), jnp.int32),
                       pltpu.VMEM((CSL,), f32)],
    )(functools.partial(_sc_counts, NM))
    c_flat = sc_counts(gm_src2, gm_dst2, zc)
    if isinstance(c_flat, (list, tuple)):
        c_flat = c_flat[0]

    sc_adj = functools.partial(
        pl.kernel, mesh=mesh,
        compiler_params=pltpu.CompilerParams(needs_layout_passes=False,
                                             use_tc_tiling_on_sc=True),
        out_type=[jax.ShapeDtypeStruct((NDP, NDP), f32)],
        scratch_types=[pltpu.VMEM((CHD, 128), jnp.int32),
                       pltpu.VMEM((CHD, 128), jnp.int32),
                       pltpu.VMEM((AROWS, NDP), f32)],
    )(_sc_adj)
    amat = sc_adj(dd_src2, dd_dst2, za)
    if isinstance(amat, (list, tuple)):
        amat = amat[0]

    cmat = c_flat[:NG * NM].reshape(NG, NM)

    full = lambda shape: pl.BlockSpec(shape, lambda i: tuple(0 for _ in shape))
    h2 = pl.pallas_call(
        _tc_drugmlp_body,
        grid=(_cdiv(ND, RD),),
        in_specs=[pl.BlockSpec((RD, DFD), lambda i: (i, 0)),
                  full((DFD, HID)), full((1, HID)), full((1, HID)), full((1, HID)),
                  full((HID, HID)), full((1, HID)), full((1, HID)), full((1, HID))],
        out_specs=pl.BlockSpec((RD, HID), lambda i: (i, 0)),
        out_shape=jax.ShapeDtypeStruct((ND, HID), f32),
    )(drug_feat, p['dW1'].astype(bf16), v2(p['db1']), v2(p['dln1_g']),
      v2(p['dln1_b']), p['dW2'].astype(bf16), v2(p['db2']), v2(p['dln2_g']),
      v2(p['dln2_b']))
    h2p = jnp.pad(h2, ((0, NDP - ND), (0, 0)))

    demb = pl.pallas_call(
        _tc_drugout_body,
        grid=(_cdiv(ND, RD),),
        in_specs=[pl.BlockSpec((RD, NDP), lambda i: (i, 0)),
                  full((NDP, HID)),
                  pl.BlockSpec((RD, HID), lambda i: (i, 0)),
                  full((HID, HID)), full((1, HID)), full((HID, HID)),
                  full((1, HID)), full((1, HID)), full((HID, OUT)), full((1, OUT))],
        out_specs=pl.BlockSpec((RD, OUT), lambda i: (i, 0)),
        out_shape=jax.ShapeDtypeStruct((ND, OUT), f32),
    )(amat, h2p, h2p,
      p['sWl'], v2(p['sbl']), p['sWr'], v2(p['sln_g']), v2(p['sln_b']),
      p['doW'], v2(p['dob']))

    hgc, mnew_un, den = pl.pallas_call(
        functools.partial(_tc_gene_body, H, C, NG),
        grid=(_cdiv(NG, RG),),
        in_specs=[pl.BlockSpec((RG, GFD), lambda i: (i, 0)),
                  pl.BlockSpec((RG, NM), lambda i: (i, 0)),
                  full((NM, MFD)),
                  full((GFD, HID)), full((1, HID)), full((1, HID)), full((1, HID)),
                  full((MFD, HID)), full((1, HID)), full((1, HID)), full((1, HID)),
                  full((HID, HID)), full((1, H * C)), full((1, H * C)),
                  full((HID, HID)), full((1, H * C)), full((1, H * C)),
                  full((1, HID)), full((1, HID)), full((1, HID))],
        out_specs=[pl.BlockSpec((RG, HID + 128), lambda i: (i, 0)),
                   full((NM, HID)), full((NM, 128))],
        out_shape=[jax.ShapeDtypeStruct((NG, HID + 128), f32),
                   jax.ShapeDtypeStruct((NM, HID), f32),
                   jax.ShapeDtypeStruct((NM, 128), f32)],
    )(gene_feat, cmat, mech_feat,
      p['gW'].astype(bf16), v2(p['gb']), v2(p['g_ln_g']), v2(p['g_ln_b']),
      p['mW'], v2(p['mb']), v2(p['m_ln_g']), v2(p['m_ln_b']),
      p['g2m0_W'].astype(bf16), v2(p['g2m0_as']), v2(p['g2m0_ad']),
      p['m2g0_W'].astype(bf16), v2(p['m2g0_as']), v2(p['m2g0_ad']),
      v2(p['m2g0_b']), v2(p['lng0_g']), v2(p['lng0_b']))

    sc_gather = functools.partial(
        pl.kernel, mesh=mesh,
        compiler_params=pltpu.CompilerParams(needs_layout_passes=False,
                                             use_tc_tiling_on_sc=True),
        out_type=[jax.ShapeDtypeStruct((B, HID + 128), f32),
                  jax.ShapeDtypeStruct((B, OUT), f32)],
        scratch_types=[pltpu.VMEM((kb, 128), jnp.int32),
                       pltpu.VMEM((kb, 128), jnp.int32),
                       pltpu.VMEM((128, HID + 128), f32),
                       pltpu.VMEM((128, OUT), f32),
                       pltpu.SemaphoreType.DMA],
    )(_sc_gather)
    geneh, de = sc_gather(hgc, demb, gidx3, didx3)

    scores = pl.pallas_call(
        functools.partial(_tc_decode_body, H, C),
        grid=(B // P,),
        in_specs=[pl.BlockSpec((P, HID + 128), lambda i: (i, 0)),
                  pl.BlockSpec((P, OUT), lambda i: (i, 0)),
                  full((NM, HID)), full((NM, 128)), full((NM, MFD)),
                  full((MFD, HID)), full((1, HID)), full((1, HID)), full((1, HID)),
                  full((1, HID)), full((1, HID)), full((1, HID)),
                  full((HID, HID)), full((1, H * C)), full((1, H * C)),
                  full((1, HID)), full((1, HID)), full((1, HID)),
                  full((HID, OUT)), full((1, OUT)), full((NM, OUT)),
                  full((2 * OUT, OUT)), full((1, OUT)), full((1, OUT)),
                  full((1, NM)), full((OUT, OUT))],
        out_specs=pl.BlockSpec((P,), lambda i: (i,)),
        out_shape=jax.ShapeDtypeStruct((B,), f32),
    )(geneh, de, mnew_un, den, mech_feat,
      p['mW'], v2(p['mb']), v2(p['m_ln_g']), v2(p['m_ln_b']),
      v2(p['g2m0_b']), v2(p['lnm0_g']), v2(p['lnm0_b']),
      p['m2g1_W'], v2(p['m2g1_as']), v2(p['m2g1_ad']), v2(p['m2g1_b']),
      v2(p['lng1_g']), v2(p['lng1_b']),
      p['goW'], v2(p['gob']), p['protos'],
      p['gmW1'], v2(p['gmb1']), p['gmW2'].reshape(1, OUT),
      jnp.broadcast_to(p['gmb2'].reshape(1, 1), (1, NM)), p['Wbil'])

    return scores


# speedup vs baseline: 56.8501x; 1.2218x over previous
import functools

import jax
import jax.numpy as jnp
from jax import lax
from jax.experimental import pallas as pl
from jax.experimental.pallas import tpu as pltpu
from jax.experimental.pallas import tpu_sc as plsc

NC = 2
NS = 16
NW = NC * NS
LEAK = 0.2


def _cdiv(a, b):
    return (a + b - 1) // b


def _ln(x, g, b):
    m = x.mean(-1, keepdims=True)
    v = ((x - m) ** 2).mean(-1, keepdims=True)
    return (x - m) / jnp.sqrt(v + 1e-5) * g + b


def _leaky(x):
    return jnp.where(x >= 0, x, LEAK * x)


def _mm(a, b):
    return jnp.dot(a, b, preferred_element_type=jnp.float32)


def _iota2(shape, d):
    return lax.broadcasted_iota(jnp.int32, shape, d)


def _sel(shape, fn):
    return fn(_iota2(shape, 0), _iota2(shape, 1)).astype(jnp.float32)


def _flat_mh(a, T4, R8):
    return (R8 * jnp.dot(a, T4, preferred_element_type=jnp.float32)
            ).sum(0, keepdims=True)


def _sc_counts(nm, gm_src2, gm_dst2, zc,
               c_out,
               src_v, dst_v, acc_c):
    rg = gm_src2.shape[0]
    chg = src_v.shape[0]
    csl = acc_c.shape[0]
    cid = lax.axis_index("c")
    sid = lax.axis_index("s")
    wid = sid * NC + cid
    ones16 = jnp.full((16,), 1.0, jnp.float32)
    lo_c = wid * csl
    pltpu.sync_copy(zc.at[pl.ds(pl.multiple_of(lo_c, 8), csl)], acc_c)
    for t in range(rg // chg):
        pltpu.sync_copy(gm_src2.at[pl.ds(t * chg, chg)], src_v)
        pltpu.sync_copy(gm_dst2.at[pl.ds(t * chg, chg)], dst_v)

        def row(r, c2):
            for j in range(128 // 16):
                s = src_v[r, pl.ds(j * 16, 16)]
                d = dst_v[r, pl.ds(j * 16, 16)]
                loc = s * nm + d - lo_c
                msk = (loc >= 0) & (loc < csl)
                locc = jnp.clip(loc, 0, csl - 1)
                plsc.addupdate_scatter(acc_c, [locc], ones16, mask=msk)
            return c2

        lax.fori_loop(0, chg, row, 0)
    pltpu.sync_copy(acc_c, c_out.at[pl.ds(pl.multiple_of(lo_c, 8), csl)])


def _sc_adj(dd_src2, dd_dst2, za,
            a_out,
            dsrc_v, ddst_v, acc_a):
    rd = dd_src2.shape[0]
    chd = dsrc_v.shape[0]
    arows = acc_a.shape[0]
    cid = lax.axis_index("c")
    sid = lax.axis_index("s")
    wid = sid * NC + cid
    ones16 = jnp.full((16,), 1.0, jnp.float32)
    for p in range(2):
        lo_r = wid * (2 * arows) + p * arows
        pltpu.sync_copy(za, acc_a)
        for t in range(rd // chd):
            pltpu.sync_copy(dd_src2.at[pl.ds(t * chd, chd)], dsrc_v)
            pltpu.sync_copy(dd_dst2.at[pl.ds(t * chd, chd)], ddst_v)

            def row2(r, c2):
                for j in range(128 // 16):
                    s = dsrc_v[r, pl.ds(j * 16, 16)]
                    d = ddst_v[r, pl.ds(j * 16, 16)]
                    rr = d - lo_r
                    msk = (rr >= 0) & (rr < arows)
                    rrc = jnp.clip(rr, 0, arows - 1)
                    plsc.addupdate_scatter(acc_a, [rrc, s], ones16, mask=msk)
                return c2

            lax.fori_loop(0, chd, row2, 0)
        pltpu.sync_copy(acc_a, a_out.at[pl.ds(pl.multiple_of(lo_r, 8), arows)])


def _sc_gather(hgc, demb, gidx3, didx3,
               geneh_out, de_out,
               gidx_v, didx_v, rows_h, rows_d, sem):
    kb = gidx_v.shape[0]
    cid = lax.axis_index("c")
    sid = lax.axis_index("s")
    wid = sid * NC + cid
    pltpu.sync_copy(gidx3.at[wid], gidx_v)
    pltpu.sync_copy(didx3.at[wid], didx_v)
    for j in range(kb):
        base = pl.multiple_of(wid * (kb * 128) + j * 128, 8)
        pltpu.async_copy(hgc.at[gidx_v.at[j]], rows_h, sem).wait()
        pltpu.sync_copy(rows_h, geneh_out.at[pl.ds(base, 128)])
        pltpu.async_copy(demb.at[didx_v.at[j]], rows_d, sem).wait()
        pltpu.sync_copy(rows_d, de_out.at[pl.ds(base, 128)])


def _tc_gene_body(H, C, NG, gf, cm, mech, gW, gb, glng, glnb, mW, mb, mlng,
                  mlnb, Wg2m, asg2m, adg2m, Wm2g, asm2g, adm2g, bm2g, lng0g,
                  lng0b, hgc_o, mnew_o, den_o):
    i = pl.program_id(0)
    NM = mech.shape[0]
    HID = gW.shape[1]
    R = gf.shape[0]
    MH = NM * H
    bf16 = jnp.bfloat16
    f32 = jnp.float32
    x = gf[...].astype(bf16)
    rowok = ((i * R + lax.broadcasted_iota(jnp.int32, (R, 1), 0)) < NG
             ).astype(f32)
    hg0 = _ln(_mm(x, gW[...]) + gb[...], glng[...], glnb[...])
    hm0 = _ln(_mm(mech[...], mW[...]) + mb[...], mlng[...], mlnb[...])
    cb = cm[...]
    G64 = _sel((HID, H), lambda r, c: (r // C) == c)
    T4 = _sel((H, MH), lambda r, c: (c % H) == r)
    T4T = _sel((MH, H), lambda r, c: (r % H) == c)
    R8 = _sel((NM, MH), lambda r, c: (c // H) == r)
    G4T = _sel((MH, NM), lambda r, c: (r // H) == c)
    M32 = _sel((MH, HID), lambda r, c: (r % H) == (c // C))
    hg0b = hg0.astype(bf16)
    hs_g = _mm(hg0b, Wg2m[...])
    es_g = _mm(hs_g * asg2m[...], G64)
    ed_m = _mm(_mm(hm0.astype(bf16), Wg2m[...]) * adg2m[...], G64)
    z32 = _leaky(_mm(es_g, T4) + _flat_mh(ed_m, T4, R8))
    cb32 = _mm(cb, R8)
    wgt = jnp.where(rowok > 0, cb32 * jnp.exp(jnp.minimum(z32, 60.0)), 0.0)
    den32 = wgt.sum(0, keepdims=True)
    den_pad = jnp.concatenate(
        [den32, jnp.zeros((1, 128 - MH), f32)], axis=1)
    res = lax.dot_general(wgt, hs_g, (((0,), (0,)), ((), ())),
                          preferred_element_type=f32)
    mnew_add = _mm(R8, res * M32)

    @pl.when(i == 0)
    def _():
        mnew_o[...] = jnp.zeros((NM, HID), f32)
        den_o[...] = jnp.zeros((NM, 128), f32)

    mnew_o[...] += mnew_add
    den_o[...] += jnp.broadcast_to(den_pad, (NM, 128))
    hs_m = _mm(hm0.astype(bf16), Wm2g[...])
    es_m = _mm(hs_m * asm2g[...], G64)
    wd = _mm(Wm2g[...].astype(f32) * adm2g[...], G64)
    ed_g = _mm(hg0, wd)
    z2 = _leaky(_flat_mh(es_m, T4, R8) + _mm(ed_g, T4))
    w2 = cb32 * jnp.exp(z2)
    al = w2 / (_mm(_mm(w2, T4T), T4) + 1e-16)
    hs2 = _mm(G4T, hs_m) * M32
    gnew = _mm(al, hs2) + bm2g[...]
    hg1 = jax.nn.gelu(_ln(gnew + hg0, lng0g[...], lng0b[...]))
    hgc_o[...] = jnp.concatenate(
        [hg1, cb, jnp.zeros((R, 128 - NM), f32)], axis=1)


def _tc_drugmlp_body(df, dW1, db1, l1g, l1b, dW2, db2, l2g, l2b, h2_o):
    x = df[...].astype(jnp.bfloat16)
    h1 = jax.nn.gelu(_ln(_mm(x, dW1[...]) + db1[...], l1g[...], l1b[...]))
    h2_o[...] = jax.nn.gelu(_ln(_mm(h1.astype(jnp.bfloat16), dW2[...])
                                + db2[...], l2g[...], l2b[...]))


def _tc_drugout_body(ablk, h2f, h2b, sWl, sbl, sWr, slng, slnb,
                     doW, dob, demb_o):
    A = ablk[...]
    deg = A.sum(-1, keepdims=True)
    msg = (_mm(A.astype(jnp.bfloat16), h2f[...].astype(jnp.bfloat16))
           / jnp.maximum(deg, 1.0))
    h = h2b[...]
    hn = _mm(msg, sWl[...]) + sbl[...] + _mm(h, sWr[...])
    hd = jax.nn.gelu(_ln(h + hn, slng[...], slnb[...]))
    demb_o[...] = _mm(hd, doW[...]) + dob[...]


def _tc_decode_body(H, C, geneh, de, mnew_un, den, mech, mW, mb, mlng,
                    mlnb, g2m0b, lnm0g, lnm0b, Wm2g1, asm2g1, adm2g1, bm2g1,
                    lng1g, lng1b, goW, gob, protos, gmW1, gmb1, gmW2v, gmb2v,
                    Wbil, sc_o):
    NM = mech.shape[0]
    HID = mW.shape[1]
    OUT = goW.shape[1]
    MH = NM * H
    f32 = jnp.float32
    G64 = _sel((HID, H), lambda r, c: (r // C) == c)
    T4 = _sel((H, MH), lambda r, c: (c % H) == r)
    T4T = _sel((MH, H), lambda r, c: (r % H) == c)
    R8 = _sel((NM, MH), lambda r, c: (c // H) == r)
    G4T = _sel((MH, NM), lambda r, c: (r // H) == c)
    M32 = _sel((MH, HID), lambda r, c: (r % H) == (c // C))
    hm0 = _ln(_mm(mech[...], mW[...]) + mb[...], mlng[...], mlnb[...])
    den_bc = jnp.broadcast_to(den[0:1, 0:MH], (NM, MH))
    dmat = _mm(R8 * den_bc, M32)
    mnew0 = mnew_un[...] / (dmat + 1e-16)
    hm1 = jax.nn.gelu(_ln(mnew0 + g2m0b[...] + hm0, lnm0g[...], lnm0b[...]))
    hs_m = _mm(hm1, Wm2g1[...])
    es_m = _mm(hs_m * asm2g1[...], G64)
    wd = _mm(Wm2g1[...] * adm2g1[...], G64)
    geh = geneh[:, 0:HID]
    cg = geneh[:, HID:HID + NM]
    ed_g = _mm(geh, wd)
    z = _leaky(_flat_mh(es_m, T4, R8) + _mm(ed_g, T4))
    w3 = _mm(cg, R8) * jnp.exp(z)
    al = w3 / (_mm(_mm(w3, T4T), T4) + 1e-16)
    hs2 = _mm(G4T, hs_m) * M32
    gnew = _mm(al, hs2) + bm2g1[...]
    hg2 = jax.nn.gelu(_ln(gnew + geh, lng1g[...], lng1b[...]))
    ge = _mm(hg2, goW[...]) + gob[...]
    u = _mm(ge, gmW1[0:OUT, :]) + gmb1[...]
    v = _mm(protos[...], gmW1[OUT:2 * OUT, :])
    w2 = gmW2v[...]
    gates = jnp.concatenate(
        [(jax.nn.gelu(u + v[m:m + 1, :]) * w2).sum(-1, keepdims=True)
         for m in range(NM)], axis=1) + gmb2v[...]
    mx = gates.max(-1, keepdims=True)
    ex = jnp.exp(gates - mx)
    w = ex / ex.sum(-1, keepdims=True)
    gfin = ge + _mm(w, protos[...])
    sc_o[...] = (_mm(gfin, Wbil[...]) * de[...]).sum(-1)


def kernel(gene_feat, mech_feat, drug_feat, params, gm_src, gm_dst,
           dd_edge_index, gene_idx, drug_idx):
    p = params
    NG, GFD = gene_feat.shape
    NM, MFD = mech_feat.shape
    ND, DFD = drug_feat.shape
    EGM = gm_src.shape[0]
    EDD = dd_edge_index.shape[1]
    B = gene_idx.shape[0]
    HID = p['gW'].shape[1]
    OUT = p['goW'].shape[1]
    H, C = p['g2m0_as'].shape
    f32 = jnp.float32
    bf16 = jnp.bfloat16

    RG = 1024
    RD = 256
    P = 512
    NDP = _cdiv(ND, 128) * 128
    CPAD = _cdiv(NG * NM + 1, NW * 8) * NW * 8
    CSL = CPAD // NW
    AROWS = NDP // (2 * NW)

    v2 = lambda a: a.reshape(1, -1)

    kgm = _cdiv(EGM, NW * 128)
    egm_p = NW * kgm * 128
    gm_src2 = jnp.pad(gm_src, (0, egm_p - EGM),
                      constant_values=NG).reshape(-1, 128).astype(jnp.int32)
    gm_dst2 = jnp.pad(gm_dst, (0, egm_p - EGM)).reshape(-1, 128).astype(jnp.int32)

    kdd = _cdiv(EDD, NW * 128)
    edd_p = NW * kdd * 128
    dd_src2 = jnp.pad(dd_edge_index[0],
                      (0, edd_p - EDD)).reshape(-1, 128).astype(jnp.int32)
    dd_dst2 = jnp.pad(dd_edge_index[1], (0, edd_p - EDD),
                      constant_values=NDP - 1).reshape(-1, 128).astype(jnp.int32)
    CHG = 80
    CHD = 64

    kb = B // (NW * 128)
    gidx3 = gene_idx.reshape(NW, kb, 128).astype(jnp.int32)
    didx3 = drug_idx.reshape(NW, kb, 128).astype(jnp.int32)

    zc = jnp.zeros((CPAD,), f32)
    za = jnp.zeros((AROWS, NDP), f32)

    mesh = plsc.VectorSubcoreMesh(core_axis_name="c", subcore_axis_name="s",
                                  num_cores=NC, num_subcores=NS)

    sc_counts = functools.partial(
        pl.kernel, mesh=mesh,
        compiler_params=pltpu.CompilerParams(needs_layout_passes=False,
                                             use_tc_tiling_on_sc=True),
        out_type=[jax.ShapeDtypeStruct((CPAD,), f32)],
        scratch_types=[pltpu.VMEM((CHG, 128), jnp.int32),
                       pltpu.VMEM((CHG, 128), jnp.int32),
                       pltpu.VMEM((CSL,), f32)],
    )(functools.partial(_sc_counts, NM))
    c_flat = sc_counts(gm_src2, gm_dst2, zc)
    if isinstance(c_flat, (list, tuple)):
        c_flat = c_flat[0]

    sc_adj = functools.partial(
        pl.kernel, mesh=mesh,
        compiler_params=pltpu.CompilerParams(needs_layout_passes=False,
                                             use_tc_tiling_on_sc=True),
        out_type=[jax.ShapeDtypeStruct((NDP, NDP), f32)],
        scratch_types=[pltpu.VMEM((CHD, 128), jnp.int32),
                       pltpu.VMEM((CHD, 128), jnp.int32),
                       pltpu.VMEM((AROWS, NDP), f32)],
    )(_sc_adj)
    amat = sc_adj(dd_src2, dd_dst2, za)
    if isinstance(amat, (list, tuple)):
        amat = amat[0]

    cmat = c_flat[:NG * NM].reshape(NG, NM)

    full = lambda shape: pl.BlockSpec(shape, lambda i: tuple(0 for _ in shape))
    h2 = pl.pallas_call(
        _tc_drugmlp_body,
        grid=(_cdiv(ND, RD),),
        in_specs=[pl.BlockSpec((RD, DFD), lambda i: (i, 0)),
                  full((DFD, HID)), full((1, HID)), full((1, HID)), full((1, HID)),
                  full((HID, HID)), full((1, HID)), full((1, HID)), full((1, HID))],
        out_specs=pl.BlockSpec((RD, HID), lambda i: (i, 0)),
        out_shape=jax.ShapeDtypeStruct((ND, HID), f32),
    )(drug_feat, p['dW1'].astype(bf16), v2(p['db1']), v2(p['dln1_g']),
      v2(p['dln1_b']), p['dW2'].astype(bf16), v2(p['db2']), v2(p['dln2_g']),
      v2(p['dln2_b']))
    h2p = jnp.pad(h2, ((0, NDP - ND), (0, 0)))

    demb = pl.pallas_call(
        _tc_drugout_body,
        grid=(_cdiv(ND, RD),),
        in_specs=[pl.BlockSpec((RD, NDP), lambda i: (i, 0)),
                  full((NDP, HID)),
                  pl.BlockSpec((RD, HID), lambda i: (i, 0)),
                  full((HID, HID)), full((1, HID)), full((HID, HID)),
                  full((1, HID)), full((1, HID)), full((HID, OUT)), full((1, OUT))],
        out_specs=pl.BlockSpec((RD, OUT), lambda i: (i, 0)),
        out_shape=jax.ShapeDtypeStruct((ND, OUT), f32),
    )(amat, h2p, h2p,
      p['sWl'], v2(p['sbl']), p['sWr'], v2(p['sln_g']), v2(p['sln_b']),
      p['doW'], v2(p['dob']))

    hgc, mnew_un, den = pl.pallas_call(
        functools.partial(_tc_gene_body, H, C, NG),
        grid=(_cdiv(NG, RG),),
        in_specs=[pl.BlockSpec((RG, GFD), lambda i: (i, 0)),
                  pl.BlockSpec((RG, NM), lambda i: (i, 0)),
                  full((NM, MFD)),
                  full((GFD, HID)), full((1, HID)), full((1, HID)), full((1, HID)),
                  full((MFD, HID)), full((1, HID)), full((1, HID)), full((1, HID)),
                  full((HID, HID)), full((1, H * C)), full((1, H * C)),
                  full((HID, HID)), full((1, H * C)), full((1, H * C)),
                  full((1, HID)), full((1, HID)), full((1, HID))],
        out_specs=[pl.BlockSpec((RG, HID + 128), lambda i: (i, 0)),
                   full((NM, HID)), full((NM, 128))],
        out_shape=[jax.ShapeDtypeStruct((NG, HID + 128), f32),
                   jax.ShapeDtypeStruct((NM, HID), f32),
                   jax.ShapeDtypeStruct((NM, 128), f32)],
    )(gene_feat, cmat, mech_feat,
      p['gW'].astype(bf16), v2(p['gb']), v2(p['g_ln_g']), v2(p['g_ln_b']),
      p['mW'], v2(p['mb']), v2(p['m_ln_g']), v2(p['m_ln_b']),
      p['g2m0_W'].astype(bf16), v2(p['g2m0_as']), v2(p['g2m0_ad']),
      p['m2g0_W'].astype(bf16), v2(p['m2g0_as']), v2(p['m2g0_ad']),
      v2(p['m2g0_b']), v2(p['lng0_g']), v2(p['lng0_b']))

    sc_gather = functools.partial(
        pl.kernel, mesh=mesh,
        compiler_params=pltpu.CompilerParams(needs_layout_passes=False,
                                             use_tc_tiling_on_sc=True),
        out_type=[jax.ShapeDtypeStruct((B, HID + 128), f32),
                  jax.ShapeDtypeStruct((B, OUT), f32)],
        scratch_types=[pltpu.VMEM((kb, 128), jnp.int32),
                       pltpu.VMEM((kb, 128), jnp.int32),
                       pltpu.VMEM((128, HID + 128), f32),
                       pltpu.VMEM((128, OUT), f32),
                       pltpu.SemaphoreType.DMA],
    )(_sc_gather)
    geneh, de = sc_gather(hgc, demb, gidx3, didx3)

    scores = pl.pallas_call(
        functools.partial(_tc_decode_body, H, C),
        grid=(B // P,),
        in_specs=[pl.BlockSpec((P, HID + 128), lambda i: (i, 0)),
                  pl.BlockSpec((P, OUT), lambda i: (i, 0)),
                  full((NM, HID)), full((NM, 128)), full((NM, MFD)),
                  full((MFD, HID)), full((1, HID)), full((1, HID)), full((1, HID)),
                  full((1, HID)), full((1, HID)), full((1, HID)),
                  full((HID, HID)), full((1, H * C)), full((1, H * C)),
                  full((1, HID)), full((1, HID)), full((1, HID)),
                  full((HID, OUT)), full((1, OUT)), full((NM, OUT)),
                  full((2 * OUT, OUT)), full((1, OUT)), full((1, OUT)),
                  full((1, NM)), full((OUT, OUT))],
        out_specs=pl.BlockSpec((P,), lambda i: (i,)),
        out_shape=jax.ShapeDtypeStruct((B,), f32),
    )(geneh, de, mnew_un, den, mech_feat,
      p['mW'], v2(p['mb']), v2(p['m_ln_g']), v2(p['m_ln_b']),
      v2(p['g2m0_b']), v2(p['lnm0_g']), v2(p['lnm0_b']),
      p['m2g1_W'], v2(p['m2g1_as']), v2(p['m2g1_ad']), v2(p['m2g1_b']),
      v2(p['lng1_g']), v2(p['lng1_b']),
      p['goW'], v2(p['gob']), p['protos'],
      p['gmW1'], v2(p['gmb1']), p['gmW2'].reshape(1, OUT),
      jnp.broadcast_to(p['gmb2'].reshape(1, 1), (1, NM)), p['Wbil'])

    return scores
